# Initial kernel scaffold; baseline (speedup 1.0000x reference)
import jax
import jax.numpy as jnp
from jax.experimental import pallas as pl


def kernel(x, edge_index, edge_type, W1, root1, b1, W2, root2, b2):
    raise NotImplementedError("write your pallas kernel here")


# speedup vs baseline: 9.3889x; 9.3889x over previous
import functools

import jax
import jax.numpy as jnp
from jax import lax
from jax.experimental import pallas as pl
from jax.experimental.pallas import tpu as pltpu
from jax.experimental.pallas import tpu_sc as plsc

N = 10000
E = 320000
R = 8
D = 128

NC = 2
NS = 16
NW = NC * NS
L = 16

EPW = E // NW
K = 80
NCH = EPW // K
KV = K // L

INVSZ = R * N + L
NPAD = 10240
RPT = NPAD // NS
RZC = 128

BN = 1000
GN = N // BN


def _mesh():
    return plsc.VectorSubcoreMesh(
        core_axis_name="c", subcore_axis_name="s",
        num_cores=NC, num_subcores=NS)


@functools.partial(
    pl.kernel,
    out_type=jax.ShapeDtypeStruct((NC * NPAD, D), jnp.float32),
    mesh=_mesh(),
    scratch_types=[
        pltpu.VMEM((K,), jnp.int32),
        pltpu.VMEM((K,), jnp.int32),
        pltpu.VMEM((K, D), jnp.float32),
        pltpu.VMEM_SHARED((R, D), jnp.float32),
        pltpu.VMEM_SHARED((NPAD, D), jnp.float32),
        pltpu.SemaphoreType.DMA,
    ],
)
def _hist_kernel(et_hbm, dst_hbm, oh_hbm, zrows_hbm, out_hbm,
                 et_v, dst_v, oh_rows_v, oh_sh, hist_sh, sem):
    c = lax.axis_index("c")
    s = lax.axis_index("s")
    wid = c * NS + s

    @pl.when(s == 0)
    def _():
        pltpu.sync_copy(oh_hbm, oh_sh)
    for z in range(RPT // RZC):
        pltpu.sync_copy(zrows_hbm, hist_sh.at[pl.ds(s * RPT + z * RZC, RZC)])
    plsc.subcore_barrier()

    def chunk(i, carry):
        base = wid * EPW + i * K
        pltpu.sync_copy(et_hbm.at[pl.ds(base, K)], et_v)
        pltpu.sync_copy(dst_hbm.at[pl.ds(base, K)], dst_v)
        pltpu.async_copy(oh_sh.at[et_v], oh_rows_v, sem).wait()
        pltpu.sync_copy(oh_rows_v, hist_sh.at[dst_v], add=True)
        return carry
    lax.fori_loop(0, NCH, chunk, 0)

    plsc.subcore_barrier()
    for z in range(RPT // RZC):
        off = s * RPT + z * RZC
        pltpu.sync_copy(hist_sh.at[pl.ds(off, RZC)],
                        out_hbm.at[pl.ds(c * NPAD + off, RZC)])


def _inv_table(h0, h1, sel):
    def body(h0_ref, h1_ref, sel_ref, o_ref):
        tot = jnp.dot(h0_ref[...] + h1_ref[...], sel_ref[...],
                      preferred_element_type=jnp.float32)
        o_ref[...] = 1.0 / jnp.maximum(tot, 1.0)
    return pl.pallas_call(
        body,
        grid=(GN,),
        in_specs=[
            pl.BlockSpec((BN, D), lambda i: (i, 0)),
            pl.BlockSpec((BN, D), lambda i: (i, 0)),
            pl.BlockSpec((D, R), lambda i: (0, 0)),
        ],
        out_specs=pl.BlockSpec((BN, R), lambda i: (i, 0)),
        out_shape=jax.ShapeDtypeStruct((N, R), jnp.float32),
    )(h0, h1, sel)


@functools.partial(
    pl.kernel,
    out_type=(jax.ShapeDtypeStruct((E,), jnp.int32),
              jax.ShapeDtypeStruct((E, L), jnp.float32)),
    mesh=_mesh(),
    scratch_types=[
        pltpu.VMEM((K,), jnp.int32),
        pltpu.VMEM((K,), jnp.int32),
        pltpu.VMEM((K,), jnp.int32),
        pltpu.VMEM((K,), jnp.int32),
        pltpu.VMEM((K + L,), jnp.int32),
        pltpu.VMEM((K, L), jnp.float32),
        pltpu.VMEM((INVSZ,), jnp.float32),
    ],
)
def _prep_kernel(et_hbm, src_hbm, dst_hbm, inv_hbm, gidx_hbm, scale_hbm,
                 et_v, src_v, dst_v, gidx_v, sidx_v, scale_v, inv_v):
    c = lax.axis_index("c")
    s = lax.axis_index("s")
    wid = c * NS + s

    pltpu.sync_copy(inv_hbm, inv_v)

    def chunk(i, carry):
        base = wid * EPW + i * K
        pltpu.sync_copy(et_hbm.at[pl.ds(base, K)], et_v)
        pltpu.sync_copy(src_hbm.at[pl.ds(base, K)], src_v)
        pltpu.sync_copy(dst_hbm.at[pl.ds(base, K)], dst_v)

        def grp(j, carry2):
            sl = pl.ds(j * L, L)
            t = et_v[sl]
            gidx_v[sl] = t * N + src_v[sl]
            sidx_v[sl] = dst_v[sl] * R + t
            return carry2
        lax.fori_loop(0, KV, grp, 0)

        def srow(j, carry2):
            sidx = sidx_v[pl.ds(j, L)][0]
            scale_v[j, :] = jnp.full((L,), inv_v[pl.ds(sidx, L)][0])
            return carry2
        lax.fori_loop(0, K, srow, 0)

        pltpu.sync_copy(gidx_v, gidx_hbm.at[pl.ds(base, K)])
        pltpu.sync_copy(scale_v, scale_hbm.at[pl.ds(base, K)])
        return carry
    lax.fori_loop(0, NCH, chunk, 0)


@functools.partial(
    pl.kernel,
    out_type=jax.ShapeDtypeStruct((NC * NPAD, D), jnp.float32),
    mesh=_mesh(),
    scratch_types=[
        pltpu.VMEM((K,), jnp.int32),
        pltpu.VMEM((K,), jnp.int32),
        pltpu.VMEM((K, L), jnp.float32),
        pltpu.VMEM((K, D), jnp.float32),
        pltpu.VMEM_SHARED((NPAD, D), jnp.float32),
        pltpu.SemaphoreType.DMA,
    ],
)
def _edge_kernel(h_hbm, gidx_hbm, dst_hbm, scale_hbm, zrows_hbm, out_hbm,
                 gidx_v, dst_v, scale_v, rows_v, acc_sh, sem):
    c = lax.axis_index("c")
    s = lax.axis_index("s")
    wid = c * NS + s

    for z in range(RPT // RZC):
        pltpu.sync_copy(zrows_hbm, acc_sh.at[pl.ds(s * RPT + z * RZC, RZC)])
    plsc.subcore_barrier()

    def chunk(i, carry):
        base = wid * EPW + i * K
        pltpu.sync_copy(gidx_hbm.at[pl.ds(base, K)], gidx_v)
        pltpu.sync_copy(dst_hbm.at[pl.ds(base, K)], dst_v)
        pltpu.sync_copy(scale_hbm.at[pl.ds(base, K)], scale_v)
        pltpu.async_copy(h_hbm.at[gidx_v], rows_v, sem).wait()

        def srow(j, carry2):
            sv = scale_v[j, :]
            for cp in range(D // L):
                sl = pl.ds(cp * L, L)
                rows_v[j, sl] = rows_v[j, sl] * sv
            return carry2
        lax.fori_loop(0, K, srow, 0)

        pltpu.sync_copy(rows_v, acc_sh.at[dst_v], add=True)
        return carry
    lax.fori_loop(0, NCH, chunk, 0)

    plsc.subcore_barrier()
    for z in range(RPT // RZC):
        off = s * RPT + z * RZC
        pltpu.sync_copy(acc_sh.at[pl.ds(off, RZC)],
                        out_hbm.at[pl.ds(c * NPAD + off, RZC)])


def _mm1(x, wall):
    def body(x_ref, w_ref, o_ref):
        o_ref[...] = jnp.dot(x_ref[...], w_ref[0],
                             preferred_element_type=jnp.float32)
    return pl.pallas_call(
        body,
        grid=(GN, R + 1),
        in_specs=[
            pl.BlockSpec((BN, D), lambda i, r: (i, 0)),
            pl.BlockSpec((1, D, D), lambda i, r: (r, 0, 0)),
        ],
        out_specs=pl.BlockSpec((BN, D), lambda i, r: (r * GN + i, 0)),
        out_shape=jax.ShapeDtypeStruct(((R + 1) * N, D), jnp.float32),
    )(x, wall)


def _mm2(hfull1, p0, p1, b1, wall):
    def body(base_ref, p0_ref, p1_ref, b_ref, w_ref, o_ref):
        h = base_ref[...] + p0_ref[...] + p1_ref[...] + b_ref[...]
        h = jnp.maximum(h, 0.0)
        o_ref[...] = jnp.dot(h, w_ref[0], preferred_element_type=jnp.float32)
    return pl.pallas_call(
        body,
        grid=(GN, R + 1),
        in_specs=[
            pl.BlockSpec((BN, D), lambda i, r: (R * GN + i, 0)),
            pl.BlockSpec((BN, D), lambda i, r: (i, 0)),
            pl.BlockSpec((BN, D), lambda i, r: (i, 0)),
            pl.BlockSpec((1, D), lambda i, r: (0, 0)),
            pl.BlockSpec((1, D, D), lambda i, r: (r, 0, 0)),
        ],
        out_specs=pl.BlockSpec((BN, D), lambda i, r: (r * GN + i, 0)),
        out_shape=jax.ShapeDtypeStruct(((R + 1) * N, D), jnp.float32),
    )(hfull1, p0, p1, b1, wall)


def _combine(hfull2, p0, p1, b2):
    def body(base_ref, p0_ref, p1_ref, b_ref, o_ref):
        o_ref[...] = base_ref[...] + p0_ref[...] + p1_ref[...] + b_ref[...]
    return pl.pallas_call(
        body,
        grid=(GN,),
        in_specs=[
            pl.BlockSpec((BN, D), lambda i: (R * GN + i, 0)),
            pl.BlockSpec((BN, D), lambda i: (i, 0)),
            pl.BlockSpec((BN, D), lambda i: (i, 0)),
            pl.BlockSpec((1, D), lambda i: (0, 0)),
        ],
        out_specs=pl.BlockSpec((BN, D), lambda i: (i, 0)),
        out_shape=jax.ShapeDtypeStruct((N, D), jnp.float32),
    )(hfull2, p0, p1, b2)


def kernel(x, edge_index, edge_type, W1, root1, b1, W2, root2, b2):
    src = edge_index[0]
    dst = edge_index[1]
    zrows = jnp.zeros((RZC, D), jnp.float32)
    lanes = jnp.arange(D)
    oh = (lanes[None, :] == (jnp.arange(R) * L)[:, None]).astype(jnp.float32)
    sel = (lanes[:, None] == (jnp.arange(R) * L)[None, :]).astype(jnp.float32)

    hist = _hist_kernel(edge_type, dst, oh, zrows)
    inv8 = _inv_table(hist[:N], hist[NPAD:NPAD + N], sel)
    inv = jnp.concatenate(
        [inv8.reshape(R * N), jnp.zeros((L,), jnp.float32)])
    gidx, scale = _prep_kernel(edge_type, src, dst, inv)

    wall1 = jnp.concatenate([W1, root1[None]], axis=0)
    wall2 = jnp.concatenate([W2, root2[None]], axis=0)

    hfull1 = _mm1(x, wall1)
    part1 = _edge_kernel(hfull1, gidx, dst, scale, zrows)
    hfull2 = _mm2(hfull1, part1[:N], part1[NPAD:NPAD + N],
                  b1.reshape(1, D), wall2)
    part2 = _edge_kernel(hfull2, gidx, dst, scale, zrows)
    return _combine(hfull2, part2[:N], part2[NPAD:NPAD + N], b2.reshape(1, D))


# speedup vs baseline: 9.4246x; 1.0038x over previous
import functools

import jax
import jax.numpy as jnp
from jax import lax
from jax.experimental import pallas as pl
from jax.experimental.pallas import tpu as pltpu
from jax.experimental.pallas import tpu_sc as plsc

N = 10000
E = 320000
R = 8
D = 128

NC = 2
NS = 16
NW = NC * NS
L = 16

EPW = E // NW
K = 80
NCH = EPW // K
KV = K // L

INVSZ = R * N + L
NPAD = 10240
RPT = NPAD // NS
RZC = 128

BN = 1000
GN = N // BN


def _mesh():
    return plsc.VectorSubcoreMesh(
        core_axis_name="c", subcore_axis_name="s",
        num_cores=NC, num_subcores=NS)


@functools.partial(
    pl.kernel,
    out_type=jax.ShapeDtypeStruct((NC * NPAD, D), jnp.float32),
    mesh=_mesh(),
    scratch_types=[
        pltpu.VMEM((K,), jnp.int32),
        pltpu.VMEM((K,), jnp.int32),
        pltpu.VMEM((K, D), jnp.float32),
        pltpu.VMEM_SHARED((R, D), jnp.float32),
        pltpu.VMEM_SHARED((NPAD, D), jnp.float32),
        pltpu.SemaphoreType.DMA,
    ],
)
def _hist_kernel(et_hbm, dst_hbm, oh_hbm, zrows_hbm, out_hbm,
                 et_v, dst_v, oh_rows_v, oh_sh, hist_sh, sem):
    c = lax.axis_index("c")
    s = lax.axis_index("s")
    wid = c * NS + s

    @pl.when(s == 0)
    def _():
        pltpu.sync_copy(oh_hbm, oh_sh)
    for z in range(RPT // RZC):
        pltpu.sync_copy(zrows_hbm, hist_sh.at[pl.ds(s * RPT + z * RZC, RZC)])
    plsc.subcore_barrier()

    def chunk(i, carry):
        base = wid * EPW + i * K
        pltpu.sync_copy(et_hbm.at[pl.ds(base, K)], et_v)
        pltpu.sync_copy(dst_hbm.at[pl.ds(base, K)], dst_v)
        pltpu.async_copy(oh_sh.at[et_v], oh_rows_v, sem).wait()
        pltpu.sync_copy(oh_rows_v, hist_sh.at[dst_v], add=True)
        return carry
    lax.fori_loop(0, NCH, chunk, 0)

    plsc.subcore_barrier()
    for z in range(RPT // RZC):
        off = s * RPT + z * RZC
        pltpu.sync_copy(hist_sh.at[pl.ds(off, RZC)],
                        out_hbm.at[pl.ds(c * NPAD + off, RZC)])


def _invrep(h0, h1, sel):
    def body(h0_ref, h1_ref, sel_ref, o_ref):
        tot = h0_ref[...] + h1_ref[...]
        m = jnp.dot(tot, sel_ref[0], preferred_element_type=jnp.float32)
        o_ref[...] = jnp.broadcast_to(1.0 / jnp.maximum(m, 1.0), (BN, D))
    return pl.pallas_call(
        body,
        grid=(GN, R),
        in_specs=[
            pl.BlockSpec((BN, D), lambda i, r: (i, 0)),
            pl.BlockSpec((BN, D), lambda i, r: (i, 0)),
            pl.BlockSpec((1, D, 1), lambda i, r: (r, 0, 0)),
        ],
        out_specs=pl.BlockSpec((BN, D), lambda i, r: (r * GN + i, 0)),
        out_shape=jax.ShapeDtypeStruct((R * N, D), jnp.float32),
    )(h0, h1, sel)


MROW = 4 * K
CHT = E // K
BC = CHT // GN


def _meta(et2, src2, dst2):
    def body(t_ref, s_ref, d_ref, o_ref):
        t = t_ref[...]
        o_ref[:, 0:K] = t * N + s_ref[...]
        o_ref[:, K:2 * K] = d_ref[...]
        o_ref[:, 2 * K:3 * K] = t * N + d_ref[...]
        o_ref[:, 3 * K:4 * K] = t
    return pl.pallas_call(
        body,
        grid=(GN,),
        in_specs=[
            pl.BlockSpec((BC, K), lambda i: (i, 0)),
            pl.BlockSpec((BC, K), lambda i: (i, 0)),
            pl.BlockSpec((BC, K), lambda i: (i, 0)),
        ],
        out_specs=pl.BlockSpec((BC, MROW), lambda i: (i, 0)),
        out_shape=jax.ShapeDtypeStruct((CHT, MROW), jnp.int32),
    )(et2, src2, dst2)


@functools.partial(
    pl.kernel,
    out_type=jax.ShapeDtypeStruct((NC * NPAD, D), jnp.float32),
    mesh=_mesh(),
    scratch_types=[
        pltpu.VMEM((MROW,), jnp.int32),
        pltpu.VMEM((K,), jnp.int32),
        pltpu.VMEM((K,), jnp.int32),
        pltpu.VMEM((K,), jnp.int32),
        pltpu.VMEM((K, D), jnp.float32),
        pltpu.VMEM((K, D), jnp.float32),
        pltpu.VMEM_SHARED((NPAD, D), jnp.float32),
        pltpu.SemaphoreType.DMA,
        pltpu.SemaphoreType.DMA,
    ],
)
def _edge_kernel(h_hbm, invrep_hbm, meta_hbm, zrows_hbm, out_hbm,
                 meta_v, gidx_v, dst_v, sidx_v, rows_v, sc_v,
                 acc_sh, semg, sems):
    c = lax.axis_index("c")
    s = lax.axis_index("s")
    wid = c * NS + s

    for z in range(RPT // RZC):
        pltpu.sync_copy(zrows_hbm, acc_sh.at[pl.ds(s * RPT + z * RZC, RZC)])
    plsc.subcore_barrier()

    def chunk(i, carry):
        row = wid * NCH + i
        pltpu.sync_copy(meta_hbm.at[pl.ds(row * MROW, MROW)], meta_v)
        for q in range(KV):
            sl = pl.ds(q * L, L)
            gidx_v[sl] = meta_v[pl.ds(q * L, L)]
            dst_v[sl] = meta_v[pl.ds(K + q * L, L)]
            sidx_v[sl] = meta_v[pl.ds(2 * K + q * L, L)]
        cg = pltpu.async_copy(h_hbm.at[gidx_v], rows_v, semg)
        cs = pltpu.async_copy(invrep_hbm.at[sidx_v], sc_v, sems)
        cg.wait()
        cs.wait()

        def srow(j, carry2):
            for cp in range(D // L):
                sl = pl.ds(cp * L, L)
                rows_v[j, sl] = rows_v[j, sl] * sc_v[j, sl]
            return carry2
        lax.fori_loop(0, K, srow, 0, unroll=2)

        pltpu.sync_copy(rows_v, acc_sh.at[dst_v], add=True)
        return carry
    lax.fori_loop(0, NCH, chunk, 0)

    plsc.subcore_barrier()
    for z in range(RPT // RZC):
        off = s * RPT + z * RZC
        pltpu.sync_copy(acc_sh.at[pl.ds(off, RZC)],
                        out_hbm.at[pl.ds(c * NPAD + off, RZC)])


def _mm1(x, wall):
    def body(x_ref, w_ref, o_ref):
        o_ref[...] = jnp.dot(x_ref[...], w_ref[0],
                             preferred_element_type=jnp.float32)
    return pl.pallas_call(
        body,
        grid=(GN, R + 1),
        in_specs=[
            pl.BlockSpec((BN, D), lambda i, r: (i, 0)),
            pl.BlockSpec((1, D, D), lambda i, r: (r, 0, 0)),
        ],
        out_specs=pl.BlockSpec((BN, D), lambda i, r: (r * GN + i, 0)),
        out_shape=jax.ShapeDtypeStruct(((R + 1) * N, D), jnp.float32),
    )(x, wall)


def _mm2(hfull1, p0, p1, b1, wall):
    def body(base_ref, p0_ref, p1_ref, b_ref, w_ref, o_ref):
        h = base_ref[...] + p0_ref[...] + p1_ref[...] + b_ref[...]
        h = jnp.maximum(h, 0.0)
        o_ref[...] = jnp.dot(h, w_ref[0], preferred_element_type=jnp.float32)
    return pl.pallas_call(
        body,
        grid=(GN, R + 1),
        in_specs=[
            pl.BlockSpec((BN, D), lambda i, r: (R * GN + i, 0)),
            pl.BlockSpec((BN, D), lambda i, r: (i, 0)),
            pl.BlockSpec((BN, D), lambda i, r: (i, 0)),
            pl.BlockSpec((1, D), lambda i, r: (0, 0)),
            pl.BlockSpec((1, D, D), lambda i, r: (r, 0, 0)),
        ],
        out_specs=pl.BlockSpec((BN, D), lambda i, r: (r * GN + i, 0)),
        out_shape=jax.ShapeDtypeStruct(((R + 1) * N, D), jnp.float32),
    )(hfull1, p0, p1, b1, wall)


def _combine(hfull2, p0, p1, b2):
    def body(base_ref, p0_ref, p1_ref, b_ref, o_ref):
        o_ref[...] = base_ref[...] + p0_ref[...] + p1_ref[...] + b_ref[...]
    return pl.pallas_call(
        body,
        grid=(GN,),
        in_specs=[
            pl.BlockSpec((BN, D), lambda i: (R * GN + i, 0)),
            pl.BlockSpec((BN, D), lambda i: (i, 0)),
            pl.BlockSpec((BN, D), lambda i: (i, 0)),
            pl.BlockSpec((1, D), lambda i: (0, 0)),
        ],
        out_specs=pl.BlockSpec((BN, D), lambda i: (i, 0)),
        out_shape=jax.ShapeDtypeStruct((N, D), jnp.float32),
    )(hfull2, p0, p1, b2)


def kernel(x, edge_index, edge_type, W1, root1, b1, W2, root2, b2):
    src = edge_index[0]
    dst = edge_index[1]
    zrows = jnp.zeros((RZC, D), jnp.float32)
    lanes = jnp.arange(D)
    oh = (lanes[None, :] == (jnp.arange(R) * L)[:, None]).astype(jnp.float32)
    sel = (lanes[None, :, None] ==
           (jnp.arange(R) * L)[:, None, None]).astype(jnp.float32)

    meta = _meta(edge_type.reshape(CHT, K), src.reshape(CHT, K),
                 dst.reshape(CHT, K)).reshape(CHT * MROW)
    hist = _hist_kernel(edge_type, dst, oh, zrows)
    invrep = _invrep(hist[:N], hist[NPAD:NPAD + N], sel)

    wall1 = jnp.concatenate([W1, root1[None]], axis=0)
    wall2 = jnp.concatenate([W2, root2[None]], axis=0)

    hfull1 = _mm1(x, wall1)
    part1 = _edge_kernel(hfull1, invrep, meta, zrows)
    hfull2 = _mm2(hfull1, part1[:N], part1[NPAD:NPAD + N],
                  b1.reshape(1, D), wall2)
    part2 = _edge_kernel(hfull2, invrep, meta, zrows)
    return _combine(hfull2, part2[:N], part2[NPAD:NPAD + N], b2.reshape(1, D))


# speedup vs baseline: 11.5283x; 1.2232x over previous
import functools

import jax
import jax.numpy as jnp
from jax import lax
from jax.experimental import pallas as pl
from jax.experimental.pallas import tpu as pltpu
from jax.experimental.pallas import tpu_sc as plsc

N = 10000
E = 320000
R = 8
D = 128

NC = 2
NS = 16
NW = NC * NS
L = 16

EPW = E // NW
K = 80
NCH = EPW // K
KV = K // L

INVSZ = R * N + L
NPAD = 10240
RPT = NPAD // NS
RZC = 128

BN = 1000
GN = N // BN


def _mesh():
    return plsc.VectorSubcoreMesh(
        core_axis_name="c", subcore_axis_name="s",
        num_cores=NC, num_subcores=NS)


@functools.partial(
    pl.kernel,
    out_type=jax.ShapeDtypeStruct((NC * NPAD, D), jnp.float32),
    mesh=_mesh(),
    scratch_types=[
        pltpu.VMEM((K,), jnp.int32),
        pltpu.VMEM((K,), jnp.int32),
        pltpu.VMEM((K, D), jnp.float32),
        pltpu.VMEM_SHARED((R, D), jnp.float32),
        pltpu.VMEM_SHARED((NPAD, D), jnp.float32),
        pltpu.SemaphoreType.DMA,
    ],
)
def _hist_kernel(et_hbm, dst_hbm, oh_hbm, zrows_hbm, out_hbm,
                 et_v, dst_v, oh_rows_v, oh_sh, hist_sh, sem):
    c = lax.axis_index("c")
    s = lax.axis_index("s")
    wid = c * NS + s

    @pl.when(s == 0)
    def _():
        pltpu.sync_copy(oh_hbm, oh_sh)
    for z in range(RPT // RZC):
        pltpu.sync_copy(zrows_hbm, hist_sh.at[pl.ds(s * RPT + z * RZC, RZC)])
    plsc.subcore_barrier()

    def chunk(i, carry):
        base = wid * EPW + i * K
        pltpu.sync_copy(et_hbm.at[pl.ds(base, K)], et_v)
        pltpu.sync_copy(dst_hbm.at[pl.ds(base, K)], dst_v)
        pltpu.async_copy(oh_sh.at[et_v], oh_rows_v, sem).wait()
        pltpu.sync_copy(oh_rows_v, hist_sh.at[dst_v], add=True)
        return carry
    lax.fori_loop(0, NCH, chunk, 0)

    plsc.subcore_barrier()
    for z in range(RPT // RZC):
        off = s * RPT + z * RZC
        pltpu.sync_copy(hist_sh.at[pl.ds(off, RZC)],
                        out_hbm.at[pl.ds(c * NPAD + off, RZC)])


def _invrep(h0, h1, sel):
    def body(h0_ref, h1_ref, sel_ref, o_ref):
        tot = h0_ref[...] + h1_ref[...]
        m = jnp.dot(tot, sel_ref[0], preferred_element_type=jnp.float32)
        o_ref[...] = jnp.broadcast_to(1.0 / jnp.maximum(m, 1.0), (BN, D))
    return pl.pallas_call(
        body,
        grid=(GN, R),
        in_specs=[
            pl.BlockSpec((BN, D), lambda i, r: (i, 0)),
            pl.BlockSpec((BN, D), lambda i, r: (i, 0)),
            pl.BlockSpec((1, D, 1), lambda i, r: (r, 0, 0)),
        ],
        out_specs=pl.BlockSpec((BN, D), lambda i, r: (r * GN + i, 0)),
        out_shape=jax.ShapeDtypeStruct((R * N, D), jnp.float32),
    )(h0, h1, sel)


MROW = 4 * K
CHT = E // K
BC = CHT // GN


def _meta(et2, src2, dst2):
    def body(t_ref, s_ref, d_ref, o_ref):
        t = t_ref[...]
        o_ref[:, 0:K] = t * N + s_ref[...]
        o_ref[:, K:2 * K] = d_ref[...]
        o_ref[:, 2 * K:3 * K] = t * N + d_ref[...]
        o_ref[:, 3 * K:4 * K] = t
    return pl.pallas_call(
        body,
        grid=(GN,),
        in_specs=[
            pl.BlockSpec((BC, K), lambda i: (i, 0)),
            pl.BlockSpec((BC, K), lambda i: (i, 0)),
            pl.BlockSpec((BC, K), lambda i: (i, 0)),
        ],
        out_specs=pl.BlockSpec((BC, MROW), lambda i: (i, 0)),
        out_shape=jax.ShapeDtypeStruct((CHT, MROW), jnp.int32),
    )(et2, src2, dst2)


@functools.partial(
    pl.kernel,
    out_type=jax.ShapeDtypeStruct((NC * NPAD, D), jnp.float32),
    mesh=_mesh(),
    scratch_types=[
        pltpu.VMEM((MROW,), jnp.int32),
        pltpu.VMEM((MROW,), jnp.int32),
        pltpu.VMEM((K,), jnp.int32),
        pltpu.VMEM((K,), jnp.int32),
        pltpu.VMEM((K,), jnp.int32),
        pltpu.VMEM((K,), jnp.int32),
        pltpu.VMEM((K,), jnp.int32),
        pltpu.VMEM((K,), jnp.int32),
        pltpu.VMEM((K, D), jnp.float32),
        pltpu.VMEM((K, D), jnp.float32),
        pltpu.VMEM((K, D), jnp.float32),
        pltpu.VMEM((K, D), jnp.float32),
        pltpu.VMEM_SHARED((NPAD, D), jnp.float32),
        pltpu.SemaphoreType.DMA,
        pltpu.SemaphoreType.DMA,
        pltpu.SemaphoreType.DMA,
        pltpu.SemaphoreType.DMA,
    ],
)
def _edge_kernel(h_hbm, invrep_hbm, meta_hbm, zrows_hbm, out_hbm,
                 meta0, meta1, gidx0, gidx1, dst0, dst1, sidx0, sidx1,
                 rows0, rows1, sc0, sc1,
                 acc_sh, semg0, sems0, semg1, sems1):
    c = lax.axis_index("c")
    s = lax.axis_index("s")
    wid = c * NS + s
    meta_v = (meta0, meta1)
    gidx_v = (gidx0, gidx1)
    dst_v = (dst0, dst1)
    sidx_v = (sidx0, sidx1)
    rows_v = (rows0, rows1)
    sc_v = (sc0, sc1)
    semg = (semg0, semg1)
    sems = (sems0, sems1)

    for z in range(RPT // RZC):
        pltpu.sync_copy(zrows_hbm, acc_sh.at[pl.ds(s * RPT + z * RZC, RZC)])
    plsc.subcore_barrier()

    def issue(i, b):
        row = wid * NCH + i
        pltpu.sync_copy(meta_hbm.at[pl.ds(row * MROW, MROW)], meta_v[b])
        for q in range(KV):
            sl = pl.ds(q * L, L)
            gidx_v[b][sl] = meta_v[b][pl.ds(q * L, L)]
            dst_v[b][sl] = meta_v[b][pl.ds(K + q * L, L)]
            sidx_v[b][sl] = meta_v[b][pl.ds(2 * K + q * L, L)]
        pltpu.async_copy(h_hbm.at[gidx_v[b]], rows_v[b], semg[b])
        pltpu.async_copy(invrep_hbm.at[sidx_v[b]], sc_v[b], sems[b])

    def finish(b):
        pltpu.make_async_copy(h_hbm.at[gidx_v[b]], rows_v[b], semg[b]).wait()
        pltpu.make_async_copy(invrep_hbm.at[sidx_v[b]],
                              sc_v[b], sems[b]).wait()

        def srow(j, carry2):
            for cp in range(D // L):
                sl = pl.ds(cp * L, L)
                rows_v[b][j, sl] = rows_v[b][j, sl] * sc_v[b][j, sl]
            return carry2
        lax.fori_loop(0, K, srow, 0, unroll=2)
        pltpu.sync_copy(rows_v[b], acc_sh.at[dst_v[b]], add=True)

    issue(0, 0)

    def pair(i, carry):
        c0 = 2 * i
        c1 = 2 * i + 1

        @pl.when(c1 < NCH)
        def _():
            issue(c1, 1)
        finish(0)

        @pl.when(c1 + 1 < NCH)
        def _():
            issue(c1 + 1, 0)

        @pl.when(c1 < NCH)
        def _():
            finish(1)
        return carry
    lax.fori_loop(0, (NCH + 1) // 2, pair, 0)

    plsc.subcore_barrier()
    for z in range(RPT // RZC):
        off = s * RPT + z * RZC
        pltpu.sync_copy(acc_sh.at[pl.ds(off, RZC)],
                        out_hbm.at[pl.ds(c * NPAD + off, RZC)])


def _mm1(x, wall):
    def body(x_ref, w_ref, o_ref):
        o_ref[...] = jnp.dot(x_ref[...], w_ref[0],
                             preferred_element_type=jnp.float32)
    return pl.pallas_call(
        body,
        grid=(GN, R + 1),
        in_specs=[
            pl.BlockSpec((BN, D), lambda i, r: (i, 0)),
            pl.BlockSpec((1, D, D), lambda i, r: (r, 0, 0)),
        ],
        out_specs=pl.BlockSpec((BN, D), lambda i, r: (r * GN + i, 0)),
        out_shape=jax.ShapeDtypeStruct(((R + 1) * N, D), jnp.float32),
    )(x, wall)


def _mm2(hfull1, p0, p1, b1, wall):
    def body(base_ref, p0_ref, p1_ref, b_ref, w_ref, o_ref):
        h = base_ref[...] + p0_ref[...] + p1_ref[...] + b_ref[...]
        h = jnp.maximum(h, 0.0)
        o_ref[...] = jnp.dot(h, w_ref[0], preferred_element_type=jnp.float32)
    return pl.pallas_call(
        body,
        grid=(GN, R + 1),
        in_specs=[
            pl.BlockSpec((BN, D), lambda i, r: (R * GN + i, 0)),
            pl.BlockSpec((BN, D), lambda i, r: (i, 0)),
            pl.BlockSpec((BN, D), lambda i, r: (i, 0)),
            pl.BlockSpec((1, D), lambda i, r: (0, 0)),
            pl.BlockSpec((1, D, D), lambda i, r: (r, 0, 0)),
        ],
        out_specs=pl.BlockSpec((BN, D), lambda i, r: (r * GN + i, 0)),
        out_shape=jax.ShapeDtypeStruct(((R + 1) * N, D), jnp.float32),
    )(hfull1, p0, p1, b1, wall)


def _combine(hfull2, p0, p1, b2):
    def body(base_ref, p0_ref, p1_ref, b_ref, o_ref):
        o_ref[...] = base_ref[...] + p0_ref[...] + p1_ref[...] + b_ref[...]
    return pl.pallas_call(
        body,
        grid=(GN,),
        in_specs=[
            pl.BlockSpec((BN, D), lambda i: (R * GN + i, 0)),
            pl.BlockSpec((BN, D), lambda i: (i, 0)),
            pl.BlockSpec((BN, D), lambda i: (i, 0)),
            pl.BlockSpec((1, D), lambda i: (0, 0)),
        ],
        out_specs=pl.BlockSpec((BN, D), lambda i: (i, 0)),
        out_shape=jax.ShapeDtypeStruct((N, D), jnp.float32),
    )(hfull2, p0, p1, b2)


def kernel(x, edge_index, edge_type, W1, root1, b1, W2, root2, b2):
    src = edge_index[0]
    dst = edge_index[1]
    zrows = jnp.zeros((RZC, D), jnp.float32)
    lanes = jnp.arange(D)
    oh = (lanes[None, :] == (jnp.arange(R) * L)[:, None]).astype(jnp.float32)
    sel = (lanes[None, :, None] ==
           (jnp.arange(R) * L)[:, None, None]).astype(jnp.float32)

    meta = _meta(edge_type.reshape(CHT, K), src.reshape(CHT, K),
                 dst.reshape(CHT, K)).reshape(CHT * MROW)
    hist = _hist_kernel(edge_type, dst, oh, zrows)
    invrep = _invrep(hist[:N], hist[NPAD:NPAD + N], sel)

    wall1 = jnp.concatenate([W1, root1[None]], axis=0)
    wall2 = jnp.concatenate([W2, root2[None]], axis=0)

    hfull1 = _mm1(x, wall1)
    part1 = _edge_kernel(hfull1, invrep, meta, zrows)
    hfull2 = _mm2(hfull1, part1[:N], part1[NPAD:NPAD + N],
                  b1.reshape(1, D), wall2)
    part2 = _edge_kernel(hfull2, invrep, meta, zrows)
    return _combine(hfull2, part2[:N], part2[NPAD:NPAD + N], b2.reshape(1, D))


# speedup vs baseline: 15.1771x; 1.3165x over previous
import functools

import jax
import jax.numpy as jnp
from jax import lax
from jax.experimental import pallas as pl
from jax.experimental.pallas import tpu as pltpu
from jax.experimental.pallas import tpu_sc as plsc

N = 10000
E = 320000
R = 8
D = 128

NC = 2
NS = 16
NW = NC * NS
L = 16

EPW = E // NW
K = 80
NCH = EPW // K
KV = K // L

INVSZ = R * N + L
NPAD = 10240
RPT = NPAD // NS
RZC = 128

BN = 1000
GN = N // BN


def _mesh():
    return plsc.VectorSubcoreMesh(
        core_axis_name="c", subcore_axis_name="s",
        num_cores=NC, num_subcores=NS)


@functools.partial(
    pl.kernel,
    out_type=jax.ShapeDtypeStruct((NC * NPAD, D), jnp.float32),
    mesh=_mesh(),
    scratch_types=[
        pltpu.VMEM((K,), jnp.int32),
        pltpu.VMEM((K,), jnp.int32),
        pltpu.VMEM((K, D), jnp.float32),
        pltpu.VMEM_SHARED((R, D), jnp.float32),
        pltpu.VMEM_SHARED((NPAD, D), jnp.float32),
        pltpu.SemaphoreType.DMA,
    ],
)
def _hist_kernel(et_hbm, dst_hbm, oh_hbm, zrows_hbm, out_hbm,
                 et_v, dst_v, oh_rows_v, oh_sh, hist_sh, sem):
    c = lax.axis_index("c")
    s = lax.axis_index("s")
    wid = c * NS + s

    @pl.when(s == 0)
    def _():
        pltpu.sync_copy(oh_hbm, oh_sh)
    for z in range(RPT // RZC):
        pltpu.sync_copy(zrows_hbm, hist_sh.at[pl.ds(s * RPT + z * RZC, RZC)])
    plsc.subcore_barrier()

    def chunk(i, carry):
        base = wid * EPW + i * K
        pltpu.sync_copy(et_hbm.at[pl.ds(base, K)], et_v)
        pltpu.sync_copy(dst_hbm.at[pl.ds(base, K)], dst_v)
        pltpu.async_copy(oh_sh.at[et_v], oh_rows_v, sem).wait()
        pltpu.sync_copy(oh_rows_v, hist_sh.at[dst_v], add=True)
        return carry
    lax.fori_loop(0, NCH, chunk, 0)

    plsc.subcore_barrier()
    for z in range(RPT // RZC):
        off = s * RPT + z * RZC
        pltpu.sync_copy(hist_sh.at[pl.ds(off, RZC)],
                        out_hbm.at[pl.ds(c * NPAD + off, RZC)])


def _invrep(h0, h1, sel):
    def body(h0_ref, h1_ref, sel_ref, o_ref):
        tot = h0_ref[...] + h1_ref[...]
        m = jnp.dot(tot, sel_ref[0], preferred_element_type=jnp.float32)
        o_ref[...] = jnp.broadcast_to(1.0 / jnp.maximum(m, 1.0), (BN, D))
    return pl.pallas_call(
        body,
        grid=(GN, R),
        in_specs=[
            pl.BlockSpec((BN, D), lambda i, r: (i, 0)),
            pl.BlockSpec((BN, D), lambda i, r: (i, 0)),
            pl.BlockSpec((1, D, 1), lambda i, r: (r, 0, 0)),
        ],
        out_specs=pl.BlockSpec((BN, D), lambda i, r: (r * GN + i, 0)),
        out_shape=jax.ShapeDtypeStruct((R * N, D), jnp.float32),
    )(h0, h1, sel)


MROW = 4 * K
CHT = E // K
BC = CHT // GN


def _meta(et2, src2, dst2):
    def body(t_ref, s_ref, d_ref, o_ref):
        t = t_ref[...]
        o_ref[:, 0:K] = t * N + s_ref[...]
        o_ref[:, K:2 * K] = d_ref[...]
        o_ref[:, 2 * K:3 * K] = t * N + d_ref[...]
        o_ref[:, 3 * K:4 * K] = t
    return pl.pallas_call(
        body,
        grid=(GN,),
        in_specs=[
            pl.BlockSpec((BC, K), lambda i: (i, 0)),
            pl.BlockSpec((BC, K), lambda i: (i, 0)),
            pl.BlockSpec((BC, K), lambda i: (i, 0)),
        ],
        out_specs=pl.BlockSpec((BC, MROW), lambda i: (i, 0)),
        out_shape=jax.ShapeDtypeStruct((CHT, MROW), jnp.int32),
    )(et2, src2, dst2)


MROW2 = K * L


@functools.partial(
    pl.kernel,
    out_type=jax.ShapeDtypeStruct((CHT * MROW2,), jnp.float32),
    mesh=_mesh(),
    scratch_types=[
        pltpu.VMEM((MROW,), jnp.int32),
        pltpu.VMEM((MROW,), jnp.int32),
        pltpu.VMEM((K, D), jnp.float32),
        pltpu.VMEM((K, D), jnp.float32),
        pltpu.VMEM((MROW2,), jnp.float32),
        pltpu.VMEM((MROW2,), jnp.float32),
        pltpu.SemaphoreType.DMA,
        pltpu.SemaphoreType.DMA,
    ],
)
def _prep_kernel(invrep_hbm, meta_hbm, out_hbm,
                 meta0, meta1, sc0, sc1, rec0, rec1, sem0, sem1):
    c = lax.axis_index("c")
    s = lax.axis_index("s")
    wid = c * NS + s
    meta_v = (meta0, meta1)
    sc_v = (sc0, sc1)
    rec_v = (rec0, rec1)
    sems = (sem0, sem1)

    def issue(i, b):
        row = wid * NCH + i
        pltpu.sync_copy(meta_hbm.at[pl.ds(row * MROW, MROW)], meta_v[b])
        pltpu.async_copy(
            invrep_hbm.at[meta_v[b].at[pl.ds(2 * K, K)]], sc_v[b], sems[b])

    def finish(i, b):
        pltpu.make_async_copy(
            invrep_hbm.at[meta_v[b].at[pl.ds(2 * K, K)]],
            sc_v[b], sems[b]).wait()

        def srow(j, carry2):
            rec_v[b][pl.ds(j * L, L)] = sc_v[b][j, pl.ds(0, L)]
            return carry2
        lax.fori_loop(0, K, srow, 0, unroll=4)
        row = wid * NCH + i
        pltpu.sync_copy(rec_v[b], out_hbm.at[pl.ds(row * MROW2, MROW2)])

    issue(0, 0)

    def pair(i, carry):
        c1 = 2 * i + 1

        @pl.when(c1 < NCH)
        def _():
            issue(c1, 1)
        finish(2 * i, 0)

        @pl.when(c1 + 1 < NCH)
        def _():
            issue(c1 + 1, 0)

        @pl.when(c1 < NCH)
        def _():
            finish(c1, 1)
        return carry
    lax.fori_loop(0, (NCH + 1) // 2, pair, 0)


@functools.partial(
    pl.kernel,
    out_type=jax.ShapeDtypeStruct((NC * NPAD, D), jnp.float32),
    mesh=_mesh(),
    scratch_types=[
        pltpu.VMEM((2 * K,), jnp.int32),
        pltpu.VMEM((2 * K,), jnp.int32),
        pltpu.VMEM((2 * K,), jnp.int32),
        pltpu.VMEM((MROW2,), jnp.float32),
        pltpu.VMEM((MROW2,), jnp.float32),
        pltpu.VMEM((MROW2,), jnp.float32),
        pltpu.VMEM((K,), jnp.int32),
        pltpu.VMEM((K,), jnp.int32),
        pltpu.VMEM((K,), jnp.int32),
        pltpu.VMEM((K,), jnp.int32),
        pltpu.VMEM((K,), jnp.int32),
        pltpu.VMEM((K,), jnp.int32),
        pltpu.VMEM((K, D), jnp.float32),
        pltpu.VMEM((K, D), jnp.float32),
        pltpu.VMEM((K, D), jnp.float32),
        pltpu.VMEM_SHARED((NPAD, D), jnp.float32),
        pltpu.SemaphoreType.DMA,
        pltpu.SemaphoreType.DMA,
        pltpu.SemaphoreType.DMA,
        pltpu.SemaphoreType.DMA,
        pltpu.SemaphoreType.DMA,
        pltpu.SemaphoreType.DMA,
    ],
)
def _edge_kernel(h_hbm, meta_hbm, rec_hbm, zrows_hbm, out_hbm,
                 meta0, meta1, meta2, rec0, rec1, rec2,
                 gidx0, gidx1, gidx2, dst0, dst1, dst2,
                 rows0, rows1, rows2, acc_sh,
                 semg0, semg1, semg2, semc0, semc1, semc2):
    c = lax.axis_index("c")
    s = lax.axis_index("s")
    wid = c * NS + s
    meta_v = (meta0, meta1, meta2)
    rec_v = (rec0, rec1, rec2)
    gidx_v = (gidx0, gidx1, gidx2)
    dst_v = (dst0, dst1, dst2)
    rows_v = (rows0, rows1, rows2)
    semg = (semg0, semg1, semg2)
    semc = (semc0, semc1, semc2)

    for z in range(RPT // RZC):
        pltpu.sync_copy(zrows_hbm, acc_sh.at[pl.ds(s * RPT + z * RZC, RZC)])
    plsc.subcore_barrier()

    def wait_scat(b):
        pltpu.make_async_copy(rows_v[b], acc_sh.at[dst_v[b]],
                              semc[b]).wait()

    def issue(i, b, scat_pending):
        if scat_pending is not False:
            @pl.when(scat_pending)
            def _():
                wait_scat(b)
        row = wid * NCH + i
        pltpu.sync_copy(meta_hbm.at[pl.ds(row * MROW, 2 * K)], meta_v[b])
        pltpu.sync_copy(rec_hbm.at[pl.ds(row * MROW2, MROW2)], rec_v[b])
        for q in range(KV):
            sl = pl.ds(q * L, L)
            gidx_v[b][sl] = meta_v[b][pl.ds(q * L, L)]
            dst_v[b][sl] = meta_v[b][pl.ds(K + q * L, L)]
        pltpu.async_copy(h_hbm.at[gidx_v[b]], rows_v[b], semg[b])

    def finish(b):
        pltpu.make_async_copy(h_hbm.at[gidx_v[b]], rows_v[b],
                              semg[b]).wait()

        def srow(j, carry2):
            sv = rec_v[b][pl.ds(j * L, L)]
            for cp in range(D // L):
                sl = pl.ds(cp * L, L)
                rows_v[b][j, sl] = rows_v[b][j, sl] * sv
            return carry2
        lax.fori_loop(0, K, srow, 0, unroll=2)
        pltpu.async_copy(rows_v[b], acc_sh.at[dst_v[b]], semc[b], add=True)

    issue(0, 0, False)
    issue(1, 1, False)

    def triple(t, carry):
        finish(0)
        issue(3 * t + 2, 2, t >= 1)
        finish(1)
        issue(3 * t + 3, 0, True)
        finish(2)
        issue(3 * t + 4, 1, True)
        return carry
    lax.fori_loop(0, (NCH - 2) // 3, triple, 0)

    finish(0)
    finish(1)
    wait_scat(2)
    wait_scat(0)
    wait_scat(1)
    plsc.subcore_barrier()
    for z in range(RPT // RZC):
        off = s * RPT + z * RZC
        pltpu.sync_copy(acc_sh.at[pl.ds(off, RZC)],
                        out_hbm.at[pl.ds(c * NPAD + off, RZC)])


def _mm1(x, wall):
    def body(x_ref, w_ref, o_ref):
        o_ref[...] = jnp.dot(x_ref[...], w_ref[0],
                             preferred_element_type=jnp.float32)
    return pl.pallas_call(
        body,
        grid=(GN, R + 1),
        in_specs=[
            pl.BlockSpec((BN, D), lambda i, r: (i, 0)),
            pl.BlockSpec((1, D, D), lambda i, r: (r, 0, 0)),
        ],
        out_specs=pl.BlockSpec((BN, D), lambda i, r: (r * GN + i, 0)),
        out_shape=jax.ShapeDtypeStruct(((R + 1) * N, D), jnp.float32),
    )(x, wall)


def _mm2(hfull1, p0, p1, b1, wall):
    def body(base_ref, p0_ref, p1_ref, b_ref, w_ref, o_ref):
        h = base_ref[...] + p0_ref[...] + p1_ref[...] + b_ref[...]
        h = jnp.maximum(h, 0.0)
        o_ref[...] = jnp.dot(h, w_ref[0], preferred_element_type=jnp.float32)
    return pl.pallas_call(
        body,
        grid=(GN, R + 1),
        in_specs=[
            pl.BlockSpec((BN, D), lambda i, r: (R * GN + i, 0)),
            pl.BlockSpec((BN, D), lambda i, r: (i, 0)),
            pl.BlockSpec((BN, D), lambda i, r: (i, 0)),
            pl.BlockSpec((1, D), lambda i, r: (0, 0)),
            pl.BlockSpec((1, D, D), lambda i, r: (r, 0, 0)),
        ],
        out_specs=pl.BlockSpec((BN, D), lambda i, r: (r * GN + i, 0)),
        out_shape=jax.ShapeDtypeStruct(((R + 1) * N, D), jnp.float32),
    )(hfull1, p0, p1, b1, wall)


def _combine(hfull2, p0, p1, b2):
    def body(base_ref, p0_ref, p1_ref, b_ref, o_ref):
        o_ref[...] = base_ref[...] + p0_ref[...] + p1_ref[...] + b_ref[...]
    return pl.pallas_call(
        body,
        grid=(GN,),
        in_specs=[
            pl.BlockSpec((BN, D), lambda i: (R * GN + i, 0)),
            pl.BlockSpec((BN, D), lambda i: (i, 0)),
            pl.BlockSpec((BN, D), lambda i: (i, 0)),
            pl.BlockSpec((1, D), lambda i: (0, 0)),
        ],
        out_specs=pl.BlockSpec((BN, D), lambda i: (i, 0)),
        out_shape=jax.ShapeDtypeStruct((N, D), jnp.float32),
    )(hfull2, p0, p1, b2)


def kernel(x, edge_index, edge_type, W1, root1, b1, W2, root2, b2):
    src = edge_index[0]
    dst = edge_index[1]
    zrows = jnp.zeros((RZC, D), jnp.float32)
    lanes = jnp.arange(D)
    oh = (lanes[None, :] == (jnp.arange(R) * L)[:, None]).astype(jnp.float32)
    sel = (lanes[None, :, None] ==
           (jnp.arange(R) * L)[:, None, None]).astype(jnp.float32)

    meta = _meta(edge_type.reshape(CHT, K), src.reshape(CHT, K),
                 dst.reshape(CHT, K)).reshape(CHT * MROW)
    hist = _hist_kernel(edge_type, dst, oh, zrows)
    invrep = _invrep(hist[:N], hist[NPAD:NPAD + N], sel)
    rec = _prep_kernel(invrep, meta)

    wall1 = jnp.concatenate([W1, root1[None]], axis=0)
    wall2 = jnp.concatenate([W2, root2[None]], axis=0)

    hfull1 = _mm1(x, wall1)
    part1 = _edge_kernel(hfull1, meta, rec, zrows)
    hfull2 = _mm2(hfull1, part1[:N], part1[NPAD:NPAD + N],
                  b1.reshape(1, D), wall2)
    part2 = _edge_kernel(hfull2, meta, rec, zrows)
    return _combine(hfull2, part2[:N], part2[NPAD:NPAD + N], b2.reshape(1, D))


# speedup vs baseline: 17.5521x; 1.1565x over previous
import functools

import jax
import jax.numpy as jnp
from jax import lax
from jax.experimental import pallas as pl
from jax.experimental.pallas import tpu as pltpu
from jax.experimental.pallas import tpu_sc as plsc

N = 10000
E = 320000
R = 8
D = 128

NC = 2
NS = 16
NW = NC * NS
L = 16

EPW = E // NW
K = 80
NCH = EPW // K
KV = K // L

INVSZ = R * N + L
NPAD = 10240
RPT = NPAD // NS
RZC = 128

BN = 1000
GN = N // BN


def _mesh():
    return plsc.VectorSubcoreMesh(
        core_axis_name="c", subcore_axis_name="s",
        num_cores=NC, num_subcores=NS)


@functools.partial(
    pl.kernel,
    out_type=jax.ShapeDtypeStruct((NC * NPAD, D), jnp.float32),
    mesh=_mesh(),
    scratch_types=[
        pltpu.VMEM((240,), jnp.int32),
        pltpu.VMEM((240,), jnp.int32),
        pltpu.VMEM((240,), jnp.int32),
        pltpu.VMEM((K,), jnp.int32),
        pltpu.VMEM((K,), jnp.int32),
        pltpu.VMEM((K,), jnp.int32),
        pltpu.VMEM((K, D), jnp.float32),
        pltpu.VMEM((K, D), jnp.float32),
        pltpu.VMEM((K, D), jnp.float32),
        pltpu.VMEM_SHARED((R, D), jnp.float32),
        pltpu.VMEM_SHARED((NPAD, D), jnp.float32),
        pltpu.SemaphoreType.DMA,
        pltpu.SemaphoreType.DMA,
        pltpu.SemaphoreType.DMA,
        pltpu.SemaphoreType.DMA,
        pltpu.SemaphoreType.DMA,
        pltpu.SemaphoreType.DMA,
    ],
)
def _hist_kernel(meta_hbm, oh_hbm, zrows_hbm, out_hbm,
                 meta0, meta1, meta2, dst0, dst1, dst2, oh0, oh1, oh2,
                 oh_sh, hist_sh,
                 semg0, semg1, semg2, semc0, semc1, semc2):
    c = lax.axis_index("c")
    s = lax.axis_index("s")
    wid = c * NS + s
    meta_v = (meta0, meta1, meta2)
    dst_v = (dst0, dst1, dst2)
    oh_v = (oh0, oh1, oh2)
    semg = (semg0, semg1, semg2)
    semc = (semc0, semc1, semc2)

    @pl.when(s == 0)
    def _():
        pltpu.sync_copy(oh_hbm, oh_sh)
    for z in range(RPT // RZC):
        pltpu.sync_copy(zrows_hbm, hist_sh.at[pl.ds(s * RPT + z * RZC, RZC)])
    plsc.subcore_barrier()

    def wait_scat(b):
        pltpu.make_async_copy(oh_v[b], hist_sh.at[dst_v[b]],
                              semc[b]).wait()

    def issue(i, b, scat_pending):
        if scat_pending is not False:
            @pl.when(scat_pending)
            def _():
                wait_scat(b)
        row = wid * NCH + i
        pltpu.sync_copy(meta_hbm.at[pl.ds(row * MROW + K, 240)], meta_v[b])
        for q in range(KV):
            dst_v[b][pl.ds(q * L, L)] = meta_v[b][pl.ds(q * L, L)]
        pltpu.async_copy(oh_sh.at[meta_v[b].at[pl.ds(2 * K, K)]],
                         oh_v[b], semg[b])

    def finish(b):
        pltpu.make_async_copy(oh_sh.at[meta_v[b].at[pl.ds(2 * K, K)]],
                              oh_v[b], semg[b]).wait()
        pltpu.async_copy(oh_v[b], hist_sh.at[dst_v[b]], semc[b], add=True)

    issue(0, 0, False)
    issue(1, 1, False)

    def triple(t, carry):
        finish(0)
        issue(3 * t + 2, 2, t >= 1)
        finish(1)
        issue(3 * t + 3, 0, True)
        finish(2)
        issue(3 * t + 4, 1, True)
        return carry
    lax.fori_loop(0, (NCH - 2) // 3, triple, 0)

    finish(0)
    finish(1)
    wait_scat(2)
    wait_scat(0)
    wait_scat(1)
    plsc.subcore_barrier()
    for z in range(RPT // RZC):
        off = s * RPT + z * RZC
        pltpu.sync_copy(hist_sh.at[pl.ds(off, RZC)],
                        out_hbm.at[pl.ds(c * NPAD + off, RZC)])


def _invrep(h0, h1, sel):
    def body(h0_ref, h1_ref, sel_ref, o_ref):
        tot = h0_ref[...] + h1_ref[...]
        m = jnp.dot(tot, sel_ref[0], preferred_element_type=jnp.float32)
        o_ref[...] = jnp.broadcast_to(1.0 / jnp.maximum(m, 1.0), (BN, D))
    return pl.pallas_call(
        body,
        grid=(GN, R),
        in_specs=[
            pl.BlockSpec((BN, D), lambda i, r: (i, 0)),
            pl.BlockSpec((BN, D), lambda i, r: (i, 0)),
            pl.BlockSpec((1, D, 1), lambda i, r: (r, 0, 0)),
        ],
        out_specs=pl.BlockSpec((BN, D), lambda i, r: (r * GN + i, 0)),
        out_shape=jax.ShapeDtypeStruct((R * N, D), jnp.float32),
    )(h0, h1, sel)


MROW = 4 * K
CHT = E // K
BC = CHT // GN


def _meta(et2, src2, dst2):
    def body(t_ref, s_ref, d_ref, o_ref):
        t = t_ref[...]
        o_ref[:, 0:K] = t * N + s_ref[...]
        o_ref[:, K:2 * K] = d_ref[...]
        o_ref[:, 2 * K:3 * K] = t * N + d_ref[...]
        o_ref[:, 3 * K:4 * K] = t
    return pl.pallas_call(
        body,
        grid=(GN,),
        in_specs=[
            pl.BlockSpec((BC, K), lambda i: (i, 0)),
            pl.BlockSpec((BC, K), lambda i: (i, 0)),
            pl.BlockSpec((BC, K), lambda i: (i, 0)),
        ],
        out_specs=pl.BlockSpec((BC, MROW), lambda i: (i, 0)),
        out_shape=jax.ShapeDtypeStruct((CHT, MROW), jnp.int32),
    )(et2, src2, dst2)


MROW2 = K * L


@functools.partial(
    pl.kernel,
    out_type=jax.ShapeDtypeStruct((CHT * MROW2,), jnp.float32),
    mesh=_mesh(),
    scratch_types=[
        pltpu.VMEM((MROW,), jnp.int32),
        pltpu.VMEM((MROW,), jnp.int32),
        pltpu.VMEM((K, D), jnp.float32),
        pltpu.VMEM((K, D), jnp.float32),
        pltpu.VMEM((MROW2,), jnp.float32),
        pltpu.VMEM((MROW2,), jnp.float32),
        pltpu.SemaphoreType.DMA,
        pltpu.SemaphoreType.DMA,
    ],
)
def _prep_kernel(invrep_hbm, meta_hbm, out_hbm,
                 meta0, meta1, sc0, sc1, rec0, rec1, sem0, sem1):
    c = lax.axis_index("c")
    s = lax.axis_index("s")
    wid = c * NS + s
    meta_v = (meta0, meta1)
    sc_v = (sc0, sc1)
    rec_v = (rec0, rec1)
    sems = (sem0, sem1)

    def issue(i, b):
        row = wid * NCH + i
        pltpu.sync_copy(meta_hbm.at[pl.ds(row * MROW, MROW)], meta_v[b])
        pltpu.async_copy(
            invrep_hbm.at[meta_v[b].at[pl.ds(2 * K, K)]], sc_v[b], sems[b])

    def finish(i, b):
        pltpu.make_async_copy(
            invrep_hbm.at[meta_v[b].at[pl.ds(2 * K, K)]],
            sc_v[b], sems[b]).wait()

        def srow(j, carry2):
            rec_v[b][pl.ds(j * L, L)] = sc_v[b][j, pl.ds(0, L)]
            return carry2
        lax.fori_loop(0, K, srow, 0, unroll=4)
        row = wid * NCH + i
        pltpu.sync_copy(rec_v[b], out_hbm.at[pl.ds(row * MROW2, MROW2)])

    issue(0, 0)

    def pair(i, carry):
        c1 = 2 * i + 1

        @pl.when(c1 < NCH)
        def _():
            issue(c1, 1)
        finish(2 * i, 0)

        @pl.when(c1 + 1 < NCH)
        def _():
            issue(c1 + 1, 0)

        @pl.when(c1 < NCH)
        def _():
            finish(c1, 1)
        return carry
    lax.fori_loop(0, (NCH + 1) // 2, pair, 0)


@functools.partial(
    pl.kernel,
    out_type=jax.ShapeDtypeStruct((NC * NPAD, D), jnp.float32),
    mesh=_mesh(),
    scratch_types=[
        pltpu.VMEM((2 * K,), jnp.int32),
        pltpu.VMEM((2 * K,), jnp.int32),
        pltpu.VMEM((2 * K,), jnp.int32),
        pltpu.VMEM((MROW2,), jnp.float32),
        pltpu.VMEM((MROW2,), jnp.float32),
        pltpu.VMEM((MROW2,), jnp.float32),
        pltpu.VMEM((K,), jnp.int32),
        pltpu.VMEM((K,), jnp.int32),
        pltpu.VMEM((K,), jnp.int32),
        pltpu.VMEM((K,), jnp.int32),
        pltpu.VMEM((K,), jnp.int32),
        pltpu.VMEM((K,), jnp.int32),
        pltpu.VMEM((K, D), jnp.float32),
        pltpu.VMEM((K, D), jnp.float32),
        pltpu.VMEM((K, D), jnp.float32),
        pltpu.VMEM_SHARED((NPAD, D), jnp.float32),
        pltpu.SemaphoreType.DMA,
        pltpu.SemaphoreType.DMA,
        pltpu.SemaphoreType.DMA,
        pltpu.SemaphoreType.DMA,
        pltpu.SemaphoreType.DMA,
        pltpu.SemaphoreType.DMA,
    ],
)
def _edge_kernel(h_hbm, meta_hbm, rec_hbm, zrows_hbm, out_hbm,
                 meta0, meta1, meta2, rec0, rec1, rec2,
                 gidx0, gidx1, gidx2, dst0, dst1, dst2,
                 rows0, rows1, rows2, acc_sh,
                 semg0, semg1, semg2, semc0, semc1, semc2):
    c = lax.axis_index("c")
    s = lax.axis_index("s")
    wid = c * NS + s
    meta_v = (meta0, meta1, meta2)
    rec_v = (rec0, rec1, rec2)
    gidx_v = (gidx0, gidx1, gidx2)
    dst_v = (dst0, dst1, dst2)
    rows_v = (rows0, rows1, rows2)
    semg = (semg0, semg1, semg2)
    semc = (semc0, semc1, semc2)

    for z in range(RPT // RZC):
        pltpu.sync_copy(zrows_hbm, acc_sh.at[pl.ds(s * RPT + z * RZC, RZC)])
    plsc.subcore_barrier()

    def wait_scat(b):
        pltpu.make_async_copy(rows_v[b], acc_sh.at[dst_v[b]],
                              semc[b]).wait()

    def issue(i, b, scat_pending):
        if scat_pending is not False:
            @pl.when(scat_pending)
            def _():
                wait_scat(b)
        row = wid * NCH + i
        pltpu.sync_copy(meta_hbm.at[pl.ds(row * MROW, 2 * K)], meta_v[b])
        pltpu.sync_copy(rec_hbm.at[pl.ds(row * MROW2, MROW2)], rec_v[b])
        for q in range(KV):
            sl = pl.ds(q * L, L)
            gidx_v[b][sl] = meta_v[b][pl.ds(q * L, L)]
            dst_v[b][sl] = meta_v[b][pl.ds(K + q * L, L)]
        pltpu.async_copy(h_hbm.at[gidx_v[b]], rows_v[b], semg[b])

    def finish(b):
        pltpu.make_async_copy(h_hbm.at[gidx_v[b]], rows_v[b],
                              semg[b]).wait()

        def srow(j, carry2):
            sv = rec_v[b][pl.ds(j * L, L)]
            for cp in range(D // L):
                sl = pl.ds(cp * L, L)
                rows_v[b][j, sl] = rows_v[b][j, sl] * sv
            return carry2
        lax.fori_loop(0, K, srow, 0, unroll=2)
        pltpu.async_copy(rows_v[b], acc_sh.at[dst_v[b]], semc[b], add=True)

    issue(0, 0, False)
    issue(1, 1, False)

    def triple(t, carry):
        finish(0)
        issue(3 * t + 2, 2, t >= 1)
        finish(1)
        issue(3 * t + 3, 0, True)
        finish(2)
        issue(3 * t + 4, 1, True)
        return carry
    lax.fori_loop(0, (NCH - 2) // 3, triple, 0)

    finish(0)
    finish(1)
    wait_scat(2)
    wait_scat(0)
    wait_scat(1)
    plsc.subcore_barrier()
    for z in range(RPT // RZC):
        off = s * RPT + z * RZC
        pltpu.sync_copy(acc_sh.at[pl.ds(off, RZC)],
                        out_hbm.at[pl.ds(c * NPAD + off, RZC)])


def _mm1(x, wall):
    def body(x_ref, w_ref, o_ref):
        o_ref[...] = jnp.dot(x_ref[...], w_ref[0],
                             preferred_element_type=jnp.float32)
    return pl.pallas_call(
        body,
        grid=(GN, R + 1),
        in_specs=[
            pl.BlockSpec((BN, D), lambda i, r: (i, 0)),
            pl.BlockSpec((1, D, D), lambda i, r: (r, 0, 0)),
        ],
        out_specs=pl.BlockSpec((BN, D), lambda i, r: (r * GN + i, 0)),
        out_shape=jax.ShapeDtypeStruct(((R + 1) * N, D), jnp.float32),
    )(x, wall)


def _mm2(hfull1, p0, p1, b1, wall):
    def body(base_ref, p0_ref, p1_ref, b_ref, w_ref, o_ref):
        h = base_ref[...] + p0_ref[...] + p1_ref[...] + b_ref[...]
        h = jnp.maximum(h, 0.0)
        o_ref[...] = jnp.dot(h, w_ref[0], preferred_element_type=jnp.float32)
    return pl.pallas_call(
        body,
        grid=(GN, R + 1),
        in_specs=[
            pl.BlockSpec((BN, D), lambda i, r: (R * GN + i, 0)),
            pl.BlockSpec((BN, D), lambda i, r: (i, 0)),
            pl.BlockSpec((BN, D), lambda i, r: (i, 0)),
            pl.BlockSpec((1, D), lambda i, r: (0, 0)),
            pl.BlockSpec((1, D, D), lambda i, r: (r, 0, 0)),
        ],
        out_specs=pl.BlockSpec((BN, D), lambda i, r: (r * GN + i, 0)),
        out_shape=jax.ShapeDtypeStruct(((R + 1) * N, D), jnp.float32),
    )(hfull1, p0, p1, b1, wall)


def _combine(hfull2, p0, p1, b2):
    def body(base_ref, p0_ref, p1_ref, b_ref, o_ref):
        o_ref[...] = base_ref[...] + p0_ref[...] + p1_ref[...] + b_ref[...]
    return pl.pallas_call(
        body,
        grid=(GN,),
        in_specs=[
            pl.BlockSpec((BN, D), lambda i: (R * GN + i, 0)),
            pl.BlockSpec((BN, D), lambda i: (i, 0)),
            pl.BlockSpec((BN, D), lambda i: (i, 0)),
            pl.BlockSpec((1, D), lambda i: (0, 0)),
        ],
        out_specs=pl.BlockSpec((BN, D), lambda i: (i, 0)),
        out_shape=jax.ShapeDtypeStruct((N, D), jnp.float32),
    )(hfull2, p0, p1, b2)


def kernel(x, edge_index, edge_type, W1, root1, b1, W2, root2, b2):
    src = edge_index[0]
    dst = edge_index[1]
    zrows = jnp.zeros((RZC, D), jnp.float32)
    lanes = jnp.arange(D)
    oh = (lanes[None, :] == (jnp.arange(R) * L)[:, None]).astype(jnp.float32)
    sel = (lanes[None, :, None] ==
           (jnp.arange(R) * L)[:, None, None]).astype(jnp.float32)

    meta = _meta(edge_type.reshape(CHT, K), src.reshape(CHT, K),
                 dst.reshape(CHT, K)).reshape(CHT * MROW)
    hist = _hist_kernel(meta, oh, zrows)
    invrep = _invrep(hist[:N], hist[NPAD:NPAD + N], sel)
    rec = _prep_kernel(invrep, meta)

    wall1 = jnp.concatenate([W1, root1[None]], axis=0)
    wall2 = jnp.concatenate([W2, root2[None]], axis=0)

    hfull1 = _mm1(x, wall1)
    part1 = _edge_kernel(hfull1, meta, rec, zrows)
    hfull2 = _mm2(hfull1, part1[:N], part1[NPAD:NPAD + N],
                  b1.reshape(1, D), wall2)
    part2 = _edge_kernel(hfull2, meta, rec, zrows)
    return _combine(hfull2, part2[:N], part2[NPAD:NPAD + N], b2.reshape(1, D))


# speedup vs baseline: 17.6800x; 1.0073x over previous
import functools

import jax
import jax.numpy as jnp
from jax import lax
from jax.experimental import pallas as pl
from jax.experimental.pallas import tpu as pltpu
from jax.experimental.pallas import tpu_sc as plsc

N = 10000
E = 320000
R = 8
D = 128

NC = 2
NS = 16
NW = NC * NS
L = 16

EPW = E // NW
K = 80
NCH = EPW // K
KV = K // L

INVSZ = R * N + L
NPAD = 10240
RPT = NPAD // NS
RZC = 640

BN = 1000
GN = N // BN


def _mesh():
    return plsc.VectorSubcoreMesh(
        core_axis_name="c", subcore_axis_name="s",
        num_cores=NC, num_subcores=NS)


@functools.partial(
    pl.kernel,
    out_type=jax.ShapeDtypeStruct((NC * NPAD, D), jnp.float32),
    mesh=_mesh(),
    scratch_types=[
        pltpu.VMEM((240,), jnp.int32),
        pltpu.VMEM((240,), jnp.int32),
        pltpu.VMEM((240,), jnp.int32),
        pltpu.VMEM((K,), jnp.int32),
        pltpu.VMEM((K,), jnp.int32),
        pltpu.VMEM((K,), jnp.int32),
        pltpu.VMEM((K, D), jnp.float32),
        pltpu.VMEM((K, D), jnp.float32),
        pltpu.VMEM((K, D), jnp.float32),
        pltpu.VMEM_SHARED((R, D), jnp.float32),
        pltpu.VMEM_SHARED((NPAD, D), jnp.float32),
        pltpu.SemaphoreType.DMA,
        pltpu.SemaphoreType.DMA,
        pltpu.SemaphoreType.DMA,
        pltpu.SemaphoreType.DMA,
        pltpu.SemaphoreType.DMA,
        pltpu.SemaphoreType.DMA,
    ],
)
def _hist_kernel(meta_hbm, oh_hbm, zrows_hbm, out_hbm,
                 meta0, meta1, meta2, dst0, dst1, dst2, oh0, oh1, oh2,
                 oh_sh, hist_sh,
                 semg0, semg1, semg2, semc0, semc1, semc2):
    c = lax.axis_index("c")
    s = lax.axis_index("s")
    wid = c * NS + s
    meta_v = (meta0, meta1, meta2)
    dst_v = (dst0, dst1, dst2)
    oh_v = (oh0, oh1, oh2)
    semg = (semg0, semg1, semg2)
    semc = (semc0, semc1, semc2)

    @pl.when(s == 0)
    def _():
        pltpu.sync_copy(oh_hbm, oh_sh)
    for z in range(RPT // RZC):
        pltpu.sync_copy(zrows_hbm, hist_sh.at[pl.ds(s * RPT + z * RZC, RZC)])
    plsc.subcore_barrier()

    def wait_scat(b):
        pltpu.make_async_copy(oh_v[b], hist_sh.at[dst_v[b]],
                              semc[b]).wait()

    def issue(i, b, scat_pending):
        if scat_pending is not False:
            @pl.when(scat_pending)
            def _():
                wait_scat(b)
        row = wid * NCH + i
        pltpu.sync_copy(meta_hbm.at[pl.ds(row * MROW + K, 240)], meta_v[b])
        for q in range(KV):
            dst_v[b][pl.ds(q * L, L)] = meta_v[b][pl.ds(q * L, L)]
        pltpu.async_copy(oh_sh.at[meta_v[b].at[pl.ds(2 * K, K)]],
                         oh_v[b], semg[b])

    def finish(b):
        pltpu.make_async_copy(oh_sh.at[meta_v[b].at[pl.ds(2 * K, K)]],
                              oh_v[b], semg[b]).wait()
        pltpu.async_copy(oh_v[b], hist_sh.at[dst_v[b]], semc[b], add=True)

    issue(0, 0, False)
    issue(1, 1, False)

    def triple(t, carry):
        finish(0)
        issue(3 * t + 2, 2, t >= 1)
        finish(1)
        issue(3 * t + 3, 0, True)
        finish(2)
        issue(3 * t + 4, 1, True)
        return carry
    lax.fori_loop(0, (NCH - 2) // 3, triple, 0)

    finish(0)
    finish(1)
    wait_scat(2)
    wait_scat(0)
    wait_scat(1)
    plsc.subcore_barrier()
    for z in range(RPT // RZC):
        off = s * RPT + z * RZC
        pltpu.sync_copy(hist_sh.at[pl.ds(off, RZC)],
                        out_hbm.at[pl.ds(c * NPAD + off, RZC)])


def _invrep(h0, h1, sel):
    def body(h0_ref, h1_ref, sel_ref, o_ref):
        tot = h0_ref[...] + h1_ref[...]
        m = jnp.dot(tot, sel_ref[0], preferred_element_type=jnp.float32)
        o_ref[...] = jnp.broadcast_to(1.0 / jnp.maximum(m, 1.0), (BN, D))
    return pl.pallas_call(
        body,
        grid=(GN, R),
        in_specs=[
            pl.BlockSpec((BN, D), lambda i, r: (i, 0)),
            pl.BlockSpec((BN, D), lambda i, r: (i, 0)),
            pl.BlockSpec((1, D, 1), lambda i, r: (r, 0, 0)),
        ],
        out_specs=pl.BlockSpec((BN, D), lambda i, r: (r * GN + i, 0)),
        out_shape=jax.ShapeDtypeStruct((R * N, D), jnp.float32),
    )(h0, h1, sel)


MROW = 4 * K
CHT = E // K
BC = CHT // GN


def _meta(et2, src2, dst2):
    def body(t_ref, s_ref, d_ref, o_ref):
        t = t_ref[...]
        o_ref[:, 0:K] = t * N + s_ref[...]
        o_ref[:, K:2 * K] = d_ref[...]
        o_ref[:, 2 * K:3 * K] = t * N + d_ref[...]
        o_ref[:, 3 * K:4 * K] = t
    return pl.pallas_call(
        body,
        grid=(GN,),
        in_specs=[
            pl.BlockSpec((BC, K), lambda i: (i, 0)),
            pl.BlockSpec((BC, K), lambda i: (i, 0)),
            pl.BlockSpec((BC, K), lambda i: (i, 0)),
        ],
        out_specs=pl.BlockSpec((BC, MROW), lambda i: (i, 0)),
        out_shape=jax.ShapeDtypeStruct((CHT, MROW), jnp.int32),
    )(et2, src2, dst2)


MROW2 = K * L


@functools.partial(
    pl.kernel,
    out_type=jax.ShapeDtypeStruct((CHT * MROW2,), jnp.float32),
    mesh=_mesh(),
    scratch_types=[
        pltpu.VMEM((MROW,), jnp.int32),
        pltpu.VMEM((MROW,), jnp.int32),
        pltpu.VMEM((K, D), jnp.float32),
        pltpu.VMEM((K, D), jnp.float32),
        pltpu.VMEM((MROW2,), jnp.float32),
        pltpu.VMEM((MROW2,), jnp.float32),
        pltpu.SemaphoreType.DMA,
        pltpu.SemaphoreType.DMA,
    ],
)
def _prep_kernel(invrep_hbm, meta_hbm, out_hbm,
                 meta0, meta1, sc0, sc1, rec0, rec1, sem0, sem1):
    c = lax.axis_index("c")
    s = lax.axis_index("s")
    wid = c * NS + s
    meta_v = (meta0, meta1)
    sc_v = (sc0, sc1)
    rec_v = (rec0, rec1)
    sems = (sem0, sem1)

    def issue(i, b):
        row = wid * NCH + i
        pltpu.sync_copy(meta_hbm.at[pl.ds(row * MROW, MROW)], meta_v[b])
        pltpu.async_copy(
            invrep_hbm.at[meta_v[b].at[pl.ds(2 * K, K)]], sc_v[b], sems[b])

    def finish(i, b):
        pltpu.make_async_copy(
            invrep_hbm.at[meta_v[b].at[pl.ds(2 * K, K)]],
            sc_v[b], sems[b]).wait()

        def srow(j, carry2):
            rec_v[b][pl.ds(j * L, L)] = sc_v[b][j, pl.ds(0, L)]
            return carry2
        lax.fori_loop(0, K, srow, 0, unroll=4)
        row = wid * NCH + i
        pltpu.sync_copy(rec_v[b], out_hbm.at[pl.ds(row * MROW2, MROW2)])

    issue(0, 0)

    def pair(i, carry):
        c1 = 2 * i + 1

        @pl.when(c1 < NCH)
        def _():
            issue(c1, 1)
        finish(2 * i, 0)

        @pl.when(c1 + 1 < NCH)
        def _():
            issue(c1 + 1, 0)

        @pl.when(c1 < NCH)
        def _():
            finish(c1, 1)
        return carry
    lax.fori_loop(0, (NCH + 1) // 2, pair, 0)


@functools.partial(
    pl.kernel,
    out_type=jax.ShapeDtypeStruct((NC * NPAD, D), jnp.float32),
    mesh=_mesh(),
    scratch_types=[
        pltpu.VMEM((2 * K,), jnp.int32),
        pltpu.VMEM((2 * K,), jnp.int32),
        pltpu.VMEM((2 * K,), jnp.int32),
        pltpu.VMEM((MROW2,), jnp.float32),
        pltpu.VMEM((MROW2,), jnp.float32),
        pltpu.VMEM((MROW2,), jnp.float32),
        pltpu.VMEM((K,), jnp.int32),
        pltpu.VMEM((K,), jnp.int32),
        pltpu.VMEM((K,), jnp.int32),
        pltpu.VMEM((K,), jnp.int32),
        pltpu.VMEM((K,), jnp.int32),
        pltpu.VMEM((K,), jnp.int32),
        pltpu.VMEM((K, D), jnp.float32),
        pltpu.VMEM((K, D), jnp.float32),
        pltpu.VMEM((K, D), jnp.float32),
        pltpu.VMEM_SHARED((NPAD, D), jnp.float32),
        pltpu.SemaphoreType.DMA,
        pltpu.SemaphoreType.DMA,
        pltpu.SemaphoreType.DMA,
        pltpu.SemaphoreType.DMA,
        pltpu.SemaphoreType.DMA,
        pltpu.SemaphoreType.DMA,
    ],
)
def _edge_kernel(h_hbm, meta_hbm, rec_hbm, zrows_hbm, out_hbm,
                 meta0, meta1, meta2, rec0, rec1, rec2,
                 gidx0, gidx1, gidx2, dst0, dst1, dst2,
                 rows0, rows1, rows2, acc_sh,
                 semg0, semg1, semg2, semc0, semc1, semc2):
    c = lax.axis_index("c")
    s = lax.axis_index("s")
    wid = c * NS + s
    meta_v = (meta0, meta1, meta2)
    rec_v = (rec0, rec1, rec2)
    gidx_v = (gidx0, gidx1, gidx2)
    dst_v = (dst0, dst1, dst2)
    rows_v = (rows0, rows1, rows2)
    semg = (semg0, semg1, semg2)
    semc = (semc0, semc1, semc2)

    for z in range(RPT // RZC):
        pltpu.sync_copy(zrows_hbm, acc_sh.at[pl.ds(s * RPT + z * RZC, RZC)])
    plsc.subcore_barrier()

    def wait_scat(b):
        pltpu.make_async_copy(rows_v[b], acc_sh.at[dst_v[b]],
                              semc[b]).wait()

    def issue(i, b, scat_pending):
        if scat_pending is not False:
            @pl.when(scat_pending)
            def _():
                wait_scat(b)
        row = wid * NCH + i
        pltpu.sync_copy(meta_hbm.at[pl.ds(row * MROW, 2 * K)], meta_v[b])
        pltpu.sync_copy(rec_hbm.at[pl.ds(row * MROW2, MROW2)], rec_v[b])
        for q in range(KV):
            sl = pl.ds(q * L, L)
            gidx_v[b][sl] = meta_v[b][pl.ds(q * L, L)]
            dst_v[b][sl] = meta_v[b][pl.ds(K + q * L, L)]
        pltpu.async_copy(h_hbm.at[gidx_v[b]], rows_v[b], semg[b])

    def finish(b):
        pltpu.make_async_copy(h_hbm.at[gidx_v[b]], rows_v[b],
                              semg[b]).wait()

        def srow(j, carry2):
            sv = rec_v[b][pl.ds(j * L, L)]
            for cp in range(D // L):
                sl = pl.ds(cp * L, L)
                rows_v[b][j, sl] = rows_v[b][j, sl] * sv
            return carry2
        lax.fori_loop(0, K, srow, 0, unroll=4)
        pltpu.async_copy(rows_v[b], acc_sh.at[dst_v[b]], semc[b], add=True)

    issue(0, 0, False)
    issue(1, 1, False)

    def triple(t, carry):
        finish(0)
        issue(3 * t + 2, 2, t >= 1)
        finish(1)
        issue(3 * t + 3, 0, True)
        finish(2)
        issue(3 * t + 4, 1, True)
        return carry
    lax.fori_loop(0, (NCH - 2) // 3, triple, 0)

    finish(0)
    finish(1)
    wait_scat(2)
    wait_scat(0)
    wait_scat(1)
    plsc.subcore_barrier()
    for z in range(RPT // RZC):
        off = s * RPT + z * RZC
        pltpu.sync_copy(acc_sh.at[pl.ds(off, RZC)],
                        out_hbm.at[pl.ds(c * NPAD + off, RZC)])


def _mm1(x, wall):
    def body(x_ref, w_ref, o_ref):
        o_ref[...] = jnp.dot(x_ref[...], w_ref[0],
                             preferred_element_type=jnp.float32)
    return pl.pallas_call(
        body,
        grid=(GN, R + 1),
        in_specs=[
            pl.BlockSpec((BN, D), lambda i, r: (i, 0)),
            pl.BlockSpec((1, D, D), lambda i, r: (r, 0, 0)),
        ],
        out_specs=pl.BlockSpec((BN, D), lambda i, r: (r * GN + i, 0)),
        out_shape=jax.ShapeDtypeStruct(((R + 1) * N, D), jnp.float32),
    )(x, wall)


def _mm2(hfull1, p0, p1, b1, wall):
    def body(base_ref, p0_ref, p1_ref, b_ref, w_ref, o_ref):
        h = base_ref[...] + p0_ref[...] + p1_ref[...] + b_ref[...]
        h = jnp.maximum(h, 0.0)
        o_ref[...] = jnp.dot(h, w_ref[0], preferred_element_type=jnp.float32)
    return pl.pallas_call(
        body,
        grid=(GN, R + 1),
        in_specs=[
            pl.BlockSpec((BN, D), lambda i, r: (R * GN + i, 0)),
            pl.BlockSpec((BN, D), lambda i, r: (i, 0)),
            pl.BlockSpec((BN, D), lambda i, r: (i, 0)),
            pl.BlockSpec((1, D), lambda i, r: (0, 0)),
            pl.BlockSpec((1, D, D), lambda i, r: (r, 0, 0)),
        ],
        out_specs=pl.BlockSpec((BN, D), lambda i, r: (r * GN + i, 0)),
        out_shape=jax.ShapeDtypeStruct(((R + 1) * N, D), jnp.float32),
    )(hfull1, p0, p1, b1, wall)


def _combine(hfull2, p0, p1, b2):
    def body(base_ref, p0_ref, p1_ref, b_ref, o_ref):
        o_ref[...] = base_ref[...] + p0_ref[...] + p1_ref[...] + b_ref[...]
    return pl.pallas_call(
        body,
        grid=(GN,),
        in_specs=[
            pl.BlockSpec((BN, D), lambda i: (R * GN + i, 0)),
            pl.BlockSpec((BN, D), lambda i: (i, 0)),
            pl.BlockSpec((BN, D), lambda i: (i, 0)),
            pl.BlockSpec((1, D), lambda i: (0, 0)),
        ],
        out_specs=pl.BlockSpec((BN, D), lambda i: (i, 0)),
        out_shape=jax.ShapeDtypeStruct((N, D), jnp.float32),
    )(hfull2, p0, p1, b2)


def kernel(x, edge_index, edge_type, W1, root1, b1, W2, root2, b2):
    src = edge_index[0]
    dst = edge_index[1]
    zrows = jnp.zeros((RZC, D), jnp.float32)
    lanes = jnp.arange(D)
    oh = (lanes[None, :] == (jnp.arange(R) * L)[:, None]).astype(jnp.float32)
    sel = (lanes[None, :, None] ==
           (jnp.arange(R) * L)[:, None, None]).astype(jnp.float32)

    meta = _meta(edge_type.reshape(CHT, K), src.reshape(CHT, K),
                 dst.reshape(CHT, K)).reshape(CHT * MROW)
    hist = _hist_kernel(meta, oh, zrows)
    invrep = _invrep(hist[:N], hist[NPAD:NPAD + N], sel)
    rec = _prep_kernel(invrep, meta)

    wall1 = jnp.concatenate([W1, root1[None]], axis=0)
    wall2 = jnp.concatenate([W2, root2[None]], axis=0)

    hfull1 = _mm1(x, wall1)
    part1 = _edge_kernel(hfull1, meta, rec, zrows)
    hfull2 = _mm2(hfull1, part1[:N], part1[NPAD:NPAD + N],
                  b1.reshape(1, D), wall2)
    part2 = _edge_kernel(hfull2, meta, rec, zrows)
    return _combine(hfull2, part2[:N], part2[NPAD:NPAD + N], b2.reshape(1, D))


# speedup vs baseline: 17.6928x; 1.0007x over previous
import functools

import jax
import jax.numpy as jnp
from jax import lax
from jax.experimental import pallas as pl
from jax.experimental.pallas import tpu as pltpu
from jax.experimental.pallas import tpu_sc as plsc

N = 10000
E = 320000
R = 8
D = 128

NC = 2
NS = 16
NW = NC * NS
L = 16

EPW = E // NW
K = 80
NCH = EPW // K
KV = K // L

NPAD = 10240
RPT = NPAD // NS
RZC = 640

BN = 1000
GN = N // BN


def _mesh():
    return plsc.VectorSubcoreMesh(
        core_axis_name="c", subcore_axis_name="s",
        num_cores=NC, num_subcores=NS)


@functools.partial(
    pl.kernel,
    out_type=jax.ShapeDtypeStruct((NC * NPAD, D), jnp.float32),
    mesh=_mesh(),
    scratch_types=[
        pltpu.VMEM((240,), jnp.int32),
        pltpu.VMEM((240,), jnp.int32),
        pltpu.VMEM((240,), jnp.int32),
        pltpu.VMEM((K,), jnp.int32),
        pltpu.VMEM((K,), jnp.int32),
        pltpu.VMEM((K,), jnp.int32),
        pltpu.VMEM((K, D), jnp.float32),
        pltpu.VMEM((K, D), jnp.float32),
        pltpu.VMEM((K, D), jnp.float32),
        pltpu.VMEM_SHARED((R, D), jnp.float32),
        pltpu.VMEM_SHARED((NPAD, D), jnp.float32),
        pltpu.SemaphoreType.DMA,
        pltpu.SemaphoreType.DMA,
        pltpu.SemaphoreType.DMA,
        pltpu.SemaphoreType.DMA,
        pltpu.SemaphoreType.DMA,
        pltpu.SemaphoreType.DMA,
    ],
)
def _hist_kernel(meta_hbm, oh_hbm, zrows_hbm, out_hbm,
                 meta0, meta1, meta2, dst0, dst1, dst2, oh0, oh1, oh2,
                 oh_sh, hist_sh,
                 semg0, semg1, semg2, semc0, semc1, semc2):
    c = lax.axis_index("c")
    s = lax.axis_index("s")
    wid = c * NS + s
    meta_v = (meta0, meta1, meta2)
    dst_v = (dst0, dst1, dst2)
    oh_v = (oh0, oh1, oh2)
    semg = (semg0, semg1, semg2)
    semc = (semc0, semc1, semc2)

    @pl.when(s == 0)
    def _():
        pltpu.sync_copy(oh_hbm, oh_sh)
    for z in range(RPT // RZC):
        pltpu.sync_copy(zrows_hbm, hist_sh.at[pl.ds(s * RPT + z * RZC, RZC)])
    plsc.subcore_barrier()

    def wait_scat(b):
        pltpu.make_async_copy(oh_v[b], hist_sh.at[dst_v[b]],
                              semc[b]).wait()

    def issue(i, b, scat_pending):
        if scat_pending is not False:
            @pl.when(scat_pending)
            def _():
                wait_scat(b)
        row = wid * NCH + i
        pltpu.sync_copy(meta_hbm.at[pl.ds(row * MROW + K, 240)], meta_v[b])
        for q in range(KV):
            dst_v[b][pl.ds(q * L, L)] = meta_v[b][pl.ds(q * L, L)]
        pltpu.async_copy(oh_sh.at[meta_v[b].at[pl.ds(2 * K, K)]],
                         oh_v[b], semg[b])

    def finish(b):
        pltpu.make_async_copy(oh_sh.at[meta_v[b].at[pl.ds(2 * K, K)]],
                              oh_v[b], semg[b]).wait()
        pltpu.async_copy(oh_v[b], hist_sh.at[dst_v[b]], semc[b], add=True)

    issue(0, 0, False)
    issue(1, 1, False)

    def triple(t, carry):
        finish(0)
        issue(3 * t + 2, 2, t >= 1)
        finish(1)
        issue(3 * t + 3, 0, True)
        finish(2)
        issue(3 * t + 4, 1, True)
        return carry
    lax.fori_loop(0, (NCH - 2) // 3, triple, 0)

    finish(0)
    finish(1)
    wait_scat(2)
    wait_scat(0)
    wait_scat(1)
    plsc.subcore_barrier()
    for z in range(RPT // RZC):
        off = s * RPT + z * RZC
        pltpu.sync_copy(hist_sh.at[pl.ds(off, RZC)],
                        out_hbm.at[pl.ds(c * NPAD + off, RZC)])


def _invrep(h0, h1, sel):
    def body(h0_ref, h1_ref, sel_ref, o_ref):
        tot = h0_ref[...] + h1_ref[...]
        m = jnp.dot(tot, sel_ref[0], preferred_element_type=jnp.float32)
        o_ref[...] = jnp.broadcast_to(1.0 / jnp.maximum(m, 1.0), (BN, D))
    return pl.pallas_call(
        body,
        grid=(GN, R),
        in_specs=[
            pl.BlockSpec((BN, D), lambda i, r: (i, 0)),
            pl.BlockSpec((BN, D), lambda i, r: (i, 0)),
            pl.BlockSpec((1, D, 1), lambda i, r: (r, 0, 0)),
        ],
        out_specs=pl.BlockSpec((BN, D), lambda i, r: (r * GN + i, 0)),
        out_shape=jax.ShapeDtypeStruct((R * N, D), jnp.float32),
    )(h0, h1, sel)


MROW = 4 * K
CHT = E // K
BC = CHT // GN


def _meta(et2, src2, dst2):
    def body(t_ref, s_ref, d_ref, o_ref):
        t = t_ref[...]
        o_ref[:, 0:K] = t * N + s_ref[...]
        o_ref[:, K:2 * K] = d_ref[...]
        o_ref[:, 2 * K:3 * K] = t * N + d_ref[...]
        o_ref[:, 3 * K:4 * K] = t
    return pl.pallas_call(
        body,
        grid=(GN,),
        in_specs=[
            pl.BlockSpec((BC, K), lambda i: (i, 0)),
            pl.BlockSpec((BC, K), lambda i: (i, 0)),
            pl.BlockSpec((BC, K), lambda i: (i, 0)),
        ],
        out_specs=pl.BlockSpec((BC, MROW), lambda i: (i, 0)),
        out_shape=jax.ShapeDtypeStruct((CHT, MROW), jnp.int32),
    )(et2, src2, dst2)


MROW2 = K * L


@functools.partial(
    pl.kernel,
    out_type=jax.ShapeDtypeStruct((CHT * MROW2,), jnp.float32),
    mesh=_mesh(),
    scratch_types=[
        pltpu.VMEM((MROW,), jnp.int32),
        pltpu.VMEM((MROW,), jnp.int32),
        pltpu.VMEM((K, D), jnp.float32),
        pltpu.VMEM((K, D), jnp.float32),
        pltpu.VMEM((MROW2,), jnp.float32),
        pltpu.VMEM((MROW2,), jnp.float32),
        pltpu.SemaphoreType.DMA,
        pltpu.SemaphoreType.DMA,
    ],
)
def _prep_kernel(invrep_hbm, meta_hbm, out_hbm,
                 meta0, meta1, sc0, sc1, rec0, rec1, sem0, sem1):
    c = lax.axis_index("c")
    s = lax.axis_index("s")
    wid = c * NS + s
    meta_v = (meta0, meta1)
    sc_v = (sc0, sc1)
    rec_v = (rec0, rec1)
    sems = (sem0, sem1)

    def issue(i, b):
        row = wid * NCH + i
        pltpu.sync_copy(meta_hbm.at[pl.ds(row * MROW, MROW)], meta_v[b])
        pltpu.async_copy(
            invrep_hbm.at[meta_v[b].at[pl.ds(2 * K, K)]], sc_v[b], sems[b])

    def finish(i, b):
        pltpu.make_async_copy(
            invrep_hbm.at[meta_v[b].at[pl.ds(2 * K, K)]],
            sc_v[b], sems[b]).wait()

        def srow(j, carry2):
            rec_v[b][pl.ds(j * L, L)] = sc_v[b][j, pl.ds(0, L)]
            return carry2
        lax.fori_loop(0, K, srow, 0, unroll=4)
        row = wid * NCH + i
        pltpu.sync_copy(rec_v[b], out_hbm.at[pl.ds(row * MROW2, MROW2)])

    issue(0, 0)

    def pair(i, carry):
        c1 = 2 * i + 1

        @pl.when(c1 < NCH)
        def _():
            issue(c1, 1)
        finish(2 * i, 0)

        @pl.when(c1 + 1 < NCH)
        def _():
            issue(c1 + 1, 0)

        @pl.when(c1 < NCH)
        def _():
            finish(c1, 1)
        return carry
    lax.fori_loop(0, (NCH + 1) // 2, pair, 0)


@functools.partial(
    pl.kernel,
    out_type=jax.ShapeDtypeStruct((NC * NPAD, D), jnp.float32),
    mesh=_mesh(),
    scratch_types=[
        pltpu.VMEM((2 * K,), jnp.int32),
        pltpu.VMEM((2 * K,), jnp.int32),
        pltpu.VMEM((2 * K,), jnp.int32),
        pltpu.VMEM((MROW2,), jnp.float32),
        pltpu.VMEM((MROW2,), jnp.float32),
        pltpu.VMEM((MROW2,), jnp.float32),
        pltpu.VMEM((K,), jnp.int32),
        pltpu.VMEM((K,), jnp.int32),
        pltpu.VMEM((K,), jnp.int32),
        pltpu.VMEM((K,), jnp.int32),
        pltpu.VMEM((K,), jnp.int32),
        pltpu.VMEM((K,), jnp.int32),
        pltpu.VMEM((K, D), jnp.float32),
        pltpu.VMEM((K, D), jnp.float32),
        pltpu.VMEM((K, D), jnp.float32),
        pltpu.VMEM_SHARED((NPAD, D), jnp.float32),
        pltpu.SemaphoreType.DMA,
        pltpu.SemaphoreType.DMA,
        pltpu.SemaphoreType.DMA,
        pltpu.SemaphoreType.DMA,
        pltpu.SemaphoreType.DMA,
        pltpu.SemaphoreType.DMA,
    ],
)
def _edge_kernel(h_hbm, meta_hbm, rec_hbm, zrows_hbm, out_hbm,
                 meta0, meta1, meta2, rec0, rec1, rec2,
                 gidx0, gidx1, gidx2, dst0, dst1, dst2,
                 rows0, rows1, rows2, acc_sh,
                 semg0, semg1, semg2, semc0, semc1, semc2):
    c = lax.axis_index("c")
    s = lax.axis_index("s")
    wid = c * NS + s
    meta_v = (meta0, meta1, meta2)
    rec_v = (rec0, rec1, rec2)
    gidx_v = (gidx0, gidx1, gidx2)
    dst_v = (dst0, dst1, dst2)
    rows_v = (rows0, rows1, rows2)
    semg = (semg0, semg1, semg2)
    semc = (semc0, semc1, semc2)

    for z in range(RPT // RZC):
        pltpu.sync_copy(zrows_hbm, acc_sh.at[pl.ds(s * RPT + z * RZC, RZC)])
    plsc.subcore_barrier()

    def wait_scat(b):
        pltpu.make_async_copy(rows_v[b], acc_sh.at[dst_v[b]],
                              semc[b]).wait()

    def issue(i, b, scat_pending):
        if scat_pending is not False:
            @pl.when(scat_pending)
            def _():
                wait_scat(b)
        row = wid * NCH + i
        pltpu.sync_copy(meta_hbm.at[pl.ds(row * MROW, 2 * K)], meta_v[b])
        pltpu.sync_copy(rec_hbm.at[pl.ds(row * MROW2, MROW2)], rec_v[b])
        for q in range(KV):
            sl = pl.ds(q * L, L)
            gidx_v[b][sl] = meta_v[b][pl.ds(q * L, L)]
            dst_v[b][sl] = meta_v[b][pl.ds(K + q * L, L)]
        pltpu.async_copy(h_hbm.at[gidx_v[b]], rows_v[b], semg[b])

    def finish(b):
        pltpu.make_async_copy(h_hbm.at[gidx_v[b]], rows_v[b],
                              semg[b]).wait()

        def srow(j, carry2):
            sv = rec_v[b][pl.ds(j * L, L)]
            for cp in range(D // L):
                sl = pl.ds(cp * L, L)
                rows_v[b][j, sl] = rows_v[b][j, sl] * sv
            return carry2
        lax.fori_loop(0, K, srow, 0, unroll=4)
        pltpu.async_copy(rows_v[b], acc_sh.at[dst_v[b]], semc[b], add=True)

    issue(0, 0, False)
    issue(1, 1, False)

    def triple(t, carry):
        finish(0)
        issue(3 * t + 2, 2, t >= 1)
        finish(1)
        issue(3 * t + 3, 0, True)
        finish(2)
        issue(3 * t + 4, 1, True)
        return carry
    lax.fori_loop(0, (NCH - 2) // 3, triple, 0)

    finish(0)
    finish(1)
    wait_scat(2)
    wait_scat(0)
    wait_scat(1)
    plsc.subcore_barrier()
    for z in range(RPT // RZC):
        off = s * RPT + z * RZC
        pltpu.sync_copy(acc_sh.at[pl.ds(off, RZC)],
                        out_hbm.at[pl.ds(c * NPAD + off, RZC)])


def _mm1(x, wall):
    def body(x_ref, w_ref, o_ref):
        o_ref[...] = jnp.dot(x_ref[...], w_ref[0],
                             preferred_element_type=jnp.float32)
    return pl.pallas_call(
        body,
        grid=(GN, R + 1),
        in_specs=[
            pl.BlockSpec((BN, D), lambda i, r: (i, 0)),
            pl.BlockSpec((1, D, D), lambda i, r: (r, 0, 0)),
        ],
        out_specs=pl.BlockSpec((BN, D), lambda i, r: (r * GN + i, 0)),
        out_shape=jax.ShapeDtypeStruct(((R + 1) * N, D), jnp.float32),
    )(x, wall)


def _mm2(hfull1, p0, p1, b1, wall):
    def body(base_ref, p0_ref, p1_ref, b_ref, w_ref, o_ref):
        h = base_ref[...] + p0_ref[...] + p1_ref[...] + b_ref[...]
        h = jnp.maximum(h, 0.0)
        o_ref[...] = jnp.dot(h, w_ref[0], preferred_element_type=jnp.float32)
    return pl.pallas_call(
        body,
        grid=(GN, R + 1),
        in_specs=[
            pl.BlockSpec((BN, D), lambda i, r: (R * GN + i, 0)),
            pl.BlockSpec((BN, D), lambda i, r: (i, 0)),
            pl.BlockSpec((BN, D), lambda i, r: (i, 0)),
            pl.BlockSpec((1, D), lambda i, r: (0, 0)),
            pl.BlockSpec((1, D, D), lambda i, r: (r, 0, 0)),
        ],
        out_specs=pl.BlockSpec((BN, D), lambda i, r: (r * GN + i, 0)),
        out_shape=jax.ShapeDtypeStruct(((R + 1) * N, D), jnp.float32),
    )(hfull1, p0, p1, b1, wall)


def _combine(hfull2, p0, p1, b2):
    def body(base_ref, p0_ref, p1_ref, b_ref, o_ref):
        o_ref[...] = base_ref[...] + p0_ref[...] + p1_ref[...] + b_ref[...]
    return pl.pallas_call(
        body,
        grid=(GN,),
        in_specs=[
            pl.BlockSpec((BN, D), lambda i: (R * GN + i, 0)),
            pl.BlockSpec((BN, D), lambda i: (i, 0)),
            pl.BlockSpec((BN, D), lambda i: (i, 0)),
            pl.BlockSpec((1, D), lambda i: (0, 0)),
        ],
        out_specs=pl.BlockSpec((BN, D), lambda i: (i, 0)),
        out_shape=jax.ShapeDtypeStruct((N, D), jnp.float32),
    )(hfull2, p0, p1, b2)


def kernel(x, edge_index, edge_type, W1, root1, b1, W2, root2, b2):
    src = edge_index[0]
    dst = edge_index[1]
    zrows = jnp.zeros((RZC, D), jnp.float32)
    lanes = jnp.arange(D)
    oh = (lanes[None, :] == (jnp.arange(R) * L)[:, None]).astype(jnp.float32)
    sel = (lanes[None, :, None] ==
           (jnp.arange(R) * L)[:, None, None]).astype(jnp.float32)

    meta = _meta(edge_type.reshape(CHT, K), src.reshape(CHT, K),
                 dst.reshape(CHT, K)).reshape(CHT * MROW)
    hist = _hist_kernel(meta, oh, zrows)
    invrep = _invrep(hist[:N], hist[NPAD:NPAD + N], sel)
    rec = _prep_kernel(invrep, meta)

    wall1 = jnp.concatenate([W1, root1[None]], axis=0)
    wall2 = jnp.concatenate([W2, root2[None]], axis=0)

    hfull1 = _mm1(x, wall1)
    part1 = _edge_kernel(hfull1, meta, rec, zrows)
    hfull2 = _mm2(hfull1, part1[:N], part1[NPAD:NPAD + N],
                  b1.reshape(1, D), wall2)
    part2 = _edge_kernel(hfull2, meta, rec, zrows)
    return _combine(hfull2, part2[:N], part2[NPAD:NPAD + N], b2.reshape(1, D))


# speedup vs baseline: 20.7313x; 1.1717x over previous
import functools

import jax
import jax.numpy as jnp
from jax import lax
from jax.experimental import pallas as pl
from jax.experimental.pallas import tpu as pltpu
from jax.experimental.pallas import tpu_sc as plsc

N = 10000
E = 320000
R = 8
D = 128

NC = 2
NS = 16
NW = NC * NS
L = 16

EPW = E // NW
K = 80
NCH = EPW // K
KV = K // L

NPAD = 10240
RPT = NPAD // NS
RZC = 640

BN = 1000
GN = N // BN


def _mesh():
    return plsc.VectorSubcoreMesh(
        core_axis_name="c", subcore_axis_name="s",
        num_cores=NC, num_subcores=NS)


@functools.partial(
    pl.kernel,
    out_type=jax.ShapeDtypeStruct((NC * NPAD, D), jnp.float32),
    mesh=_mesh(),
    scratch_types=[
        pltpu.VMEM((240,), jnp.int32),
        pltpu.VMEM((240,), jnp.int32),
        pltpu.VMEM((240,), jnp.int32),
        pltpu.VMEM((K,), jnp.int32),
        pltpu.VMEM((K,), jnp.int32),
        pltpu.VMEM((K,), jnp.int32),
        pltpu.VMEM((K, D), jnp.float32),
        pltpu.VMEM((K, D), jnp.float32),
        pltpu.VMEM((K, D), jnp.float32),
        pltpu.VMEM_SHARED((R, D), jnp.float32),
        pltpu.VMEM_SHARED((NPAD, D), jnp.float32),
        pltpu.SemaphoreType.DMA,
        pltpu.SemaphoreType.DMA,
        pltpu.SemaphoreType.DMA,
        pltpu.SemaphoreType.DMA,
        pltpu.SemaphoreType.DMA,
        pltpu.SemaphoreType.DMA,
    ],
)
def _hist_kernel(meta_hbm, oh_hbm, zrows_hbm, out_hbm,
                 meta0, meta1, meta2, dst0, dst1, dst2, oh0, oh1, oh2,
                 oh_sh, hist_sh,
                 semg0, semg1, semg2, semc0, semc1, semc2):
    c = lax.axis_index("c")
    s = lax.axis_index("s")
    wid = c * NS + s
    meta_v = (meta0, meta1, meta2)
    dst_v = (dst0, dst1, dst2)
    oh_v = (oh0, oh1, oh2)
    semg = (semg0, semg1, semg2)
    semc = (semc0, semc1, semc2)

    @pl.when(s == 0)
    def _():
        pltpu.sync_copy(oh_hbm, oh_sh)
    for z in range(RPT // RZC):
        pltpu.sync_copy(zrows_hbm, hist_sh.at[pl.ds(s * RPT + z * RZC, RZC)])
    plsc.subcore_barrier()

    def wait_scat(b):
        pltpu.make_async_copy(oh_v[b], hist_sh.at[dst_v[b]],
                              semc[b]).wait()

    def issue(i, b, scat_pending):
        if scat_pending is not False:
            @pl.when(scat_pending)
            def _():
                wait_scat(b)
        row = wid * NCH + i
        pltpu.sync_copy(meta_hbm.at[pl.ds(row * MROW + K, 240)], meta_v[b])
        for q in range(KV):
            dst_v[b][pl.ds(q * L, L)] = meta_v[b][pl.ds(q * L, L)]
        pltpu.async_copy(oh_sh.at[meta_v[b].at[pl.ds(2 * K, K)]],
                         oh_v[b], semg[b])

    def finish(b):
        pltpu.make_async_copy(oh_sh.at[meta_v[b].at[pl.ds(2 * K, K)]],
                              oh_v[b], semg[b]).wait()
        pltpu.async_copy(oh_v[b], hist_sh.at[dst_v[b]], semc[b], add=True)

    issue(0, 0, False)
    issue(1, 1, False)

    def triple(t, carry):
        finish(0)
        issue(3 * t + 2, 2, t >= 1)
        finish(1)
        issue(3 * t + 3, 0, True)
        finish(2)
        issue(3 * t + 4, 1, True)
        return carry
    lax.fori_loop(0, (NCH - 2) // 3, triple, 0)

    finish(0)
    finish(1)
    wait_scat(2)
    wait_scat(0)
    wait_scat(1)
    plsc.subcore_barrier()
    for z in range(RPT // RZC):
        off = s * RPT + z * RZC
        pltpu.sync_copy(hist_sh.at[pl.ds(off, RZC)],
                        out_hbm.at[pl.ds(c * NPAD + off, RZC)])


def _invrep(h0, h1, sel):
    def body(h0_ref, h1_ref, sel_ref, o_ref):
        tot = h0_ref[...] + h1_ref[...]
        m = jnp.dot(tot, sel_ref[0], preferred_element_type=jnp.float32)
        o_ref[...] = jnp.broadcast_to(1.0 / jnp.maximum(m, 1.0), (BN, D))
    return pl.pallas_call(
        body,
        grid=(GN, R),
        in_specs=[
            pl.BlockSpec((BN, D), lambda i, r: (i, 0)),
            pl.BlockSpec((BN, D), lambda i, r: (i, 0)),
            pl.BlockSpec((1, D, 1), lambda i, r: (r, 0, 0)),
        ],
        out_specs=pl.BlockSpec((BN, D), lambda i, r: (r * GN + i, 0)),
        out_shape=jax.ShapeDtypeStruct((R * N, D), jnp.float32),
    )(h0, h1, sel)


MROW = 4 * K
CHT = E // K
BC = CHT // GN


def _meta(et2, src2, dst2):
    def body(t_ref, s_ref, d_ref, o_ref):
        t = t_ref[...]
        o_ref[:, 0:K] = t * N + s_ref[...]
        o_ref[:, K:2 * K] = d_ref[...]
        o_ref[:, 2 * K:3 * K] = t * N + d_ref[...]
        o_ref[:, 3 * K:4 * K] = t
    return pl.pallas_call(
        body,
        grid=(GN,),
        in_specs=[
            pl.BlockSpec((BC, K), lambda i: (i, 0)),
            pl.BlockSpec((BC, K), lambda i: (i, 0)),
            pl.BlockSpec((BC, K), lambda i: (i, 0)),
        ],
        out_specs=pl.BlockSpec((BC, MROW), lambda i: (i, 0)),
        out_shape=jax.ShapeDtypeStruct((CHT, MROW), jnp.int32),
    )(et2, src2, dst2)


MROW2 = K * L


@functools.partial(
    pl.kernel,
    out_type=jax.ShapeDtypeStruct((CHT * MROW2,), jnp.float32),
    mesh=_mesh(),
    scratch_types=[
        pltpu.VMEM((MROW,), jnp.int32),
        pltpu.VMEM((MROW,), jnp.int32),
        pltpu.VMEM((K, D), jnp.float32),
        pltpu.VMEM((K, D), jnp.float32),
        pltpu.VMEM((MROW2,), jnp.float32),
        pltpu.VMEM((MROW2,), jnp.float32),
        pltpu.SemaphoreType.DMA,
        pltpu.SemaphoreType.DMA,
    ],
)
def _prep_kernel(invrep_hbm, meta_hbm, out_hbm,
                 meta0, meta1, sc0, sc1, rec0, rec1, sem0, sem1):
    c = lax.axis_index("c")
    s = lax.axis_index("s")
    wid = c * NS + s
    meta_v = (meta0, meta1)
    sc_v = (sc0, sc1)
    rec_v = (rec0, rec1)
    sems = (sem0, sem1)

    def issue(i, b):
        row = wid * NCH + i
        pltpu.sync_copy(meta_hbm.at[pl.ds(row * MROW, MROW)], meta_v[b])
        pltpu.async_copy(
            invrep_hbm.at[meta_v[b].at[pl.ds(2 * K, K)]], sc_v[b], sems[b])

    def finish(i, b):
        pltpu.make_async_copy(
            invrep_hbm.at[meta_v[b].at[pl.ds(2 * K, K)]],
            sc_v[b], sems[b]).wait()

        def srow(j, carry2):
            rec_v[b][pl.ds(j * L, L)] = sc_v[b][j, pl.ds(0, L)]
            return carry2
        lax.fori_loop(0, K, srow, 0, unroll=4)
        row = wid * NCH + i
        pltpu.sync_copy(rec_v[b], out_hbm.at[pl.ds(row * MROW2, MROW2)])

    issue(0, 0)

    def pair(i, carry):
        c1 = 2 * i + 1

        @pl.when(c1 < NCH)
        def _():
            issue(c1, 1)
        finish(2 * i, 0)

        @pl.when(c1 + 1 < NCH)
        def _():
            issue(c1 + 1, 0)

        @pl.when(c1 < NCH)
        def _():
            finish(c1, 1)
        return carry
    lax.fori_loop(0, (NCH + 1) // 2, pair, 0)


@functools.partial(
    pl.kernel,
    out_type=jax.ShapeDtypeStruct((NC * NPAD, D), jnp.float32),
    mesh=_mesh(),
    scratch_types=[
        pltpu.VMEM((2 * K,), jnp.int32),
        pltpu.VMEM((2 * K,), jnp.int32),
        pltpu.VMEM((2 * K,), jnp.int32),
        pltpu.VMEM((MROW2,), jnp.float32),
        pltpu.VMEM((MROW2,), jnp.float32),
        pltpu.VMEM((MROW2,), jnp.float32),
        pltpu.VMEM((K,), jnp.int32),
        pltpu.VMEM((K,), jnp.int32),
        pltpu.VMEM((K,), jnp.int32),
        pltpu.VMEM((K,), jnp.int32),
        pltpu.VMEM((K,), jnp.int32),
        pltpu.VMEM((K,), jnp.int32),
        pltpu.VMEM((K, D), jnp.float32),
        pltpu.VMEM((K, D), jnp.float32),
        pltpu.VMEM((K, D), jnp.float32),
        pltpu.VMEM_SHARED((NPAD, D), jnp.float32),
        pltpu.SemaphoreType.DMA,
        pltpu.SemaphoreType.DMA,
        pltpu.SemaphoreType.DMA,
        pltpu.SemaphoreType.DMA,
        pltpu.SemaphoreType.DMA,
        pltpu.SemaphoreType.DMA,
        pltpu.SemaphoreType.DMA,
        pltpu.SemaphoreType.DMA,
        pltpu.SemaphoreType.DMA,
    ],
)
def _edge_kernel(h_hbm, meta_hbm, rec_hbm, zrows_hbm, out_hbm,
                 meta0, meta1, meta2, rec0, rec1, rec2,
                 gidx0, gidx1, gidx2, dst0, dst1, dst2,
                 rows0, rows1, rows2, acc_sh,
                 semg0, semg1, semg2, semc0, semc1, semc2,
                 semr0, semr1, semr2):
    c = lax.axis_index("c")
    s = lax.axis_index("s")
    wid = c * NS + s
    meta_v = (meta0, meta1, meta2)
    rec_v = (rec0, rec1, rec2)
    gidx_v = (gidx0, gidx1, gidx2)
    dst_v = (dst0, dst1, dst2)
    rows_v = (rows0, rows1, rows2)
    semg = (semg0, semg1, semg2)
    semc = (semc0, semc1, semc2)
    semr = (semr0, semr1, semr2)

    for z in range(RPT // RZC):
        pltpu.sync_copy(zrows_hbm, acc_sh.at[pl.ds(s * RPT + z * RZC, RZC)])
    plsc.subcore_barrier()

    def wait_scat(b):
        pltpu.make_async_copy(rows_v[b], acc_sh.at[dst_v[b]],
                              semc[b]).wait()

    def issue(i, b, scat_pending):
        if scat_pending is not False:
            @pl.when(scat_pending)
            def _():
                wait_scat(b)
        row = wid * NCH + i
        pltpu.async_copy(rec_hbm.at[pl.ds(row * MROW2, MROW2)],
                         rec_v[b], semr[b])
        pltpu.sync_copy(meta_hbm.at[pl.ds(row * MROW, 2 * K)], meta_v[b])
        for q in range(KV):
            sl = pl.ds(q * L, L)
            gidx_v[b][sl] = meta_v[b][pl.ds(q * L, L)]
            dst_v[b][sl] = meta_v[b][pl.ds(K + q * L, L)]
        pltpu.async_copy(h_hbm.at[gidx_v[b]], rows_v[b], semg[b])

    def finish(b):
        pltpu.make_async_copy(rec_hbm.at[pl.ds(0, MROW2)],
                              rec_v[b], semr[b]).wait()
        pltpu.make_async_copy(h_hbm.at[gidx_v[b]], rows_v[b],
                              semg[b]).wait()

        def srow(j, carry2):
            sv = rec_v[b][pl.ds(j * L, L)]
            for cp in range(D // L):
                sl = pl.ds(cp * L, L)
                rows_v[b][j, sl] = rows_v[b][j, sl] * sv
            return carry2
        lax.fori_loop(0, K, srow, 0, unroll=4)
        pltpu.async_copy(rows_v[b], acc_sh.at[dst_v[b]], semc[b], add=True)

    issue(0, 0, False)
    issue(1, 1, False)

    def triple(t, carry):
        finish(0)
        issue(3 * t + 2, 2, t >= 1)
        finish(1)
        issue(3 * t + 3, 0, True)
        finish(2)
        issue(3 * t + 4, 1, True)
        return carry
    lax.fori_loop(0, (NCH - 2) // 3, triple, 0)

    finish(0)
    finish(1)
    wait_scat(2)
    wait_scat(0)
    wait_scat(1)
    plsc.subcore_barrier()
    for z in range(RPT // RZC):
        off = s * RPT + z * RZC
        pltpu.sync_copy(acc_sh.at[pl.ds(off, RZC)],
                        out_hbm.at[pl.ds(c * NPAD + off, RZC)])


def _mm1(x, wall):
    def body(x_ref, w_ref, o_ref):
        o_ref[...] = jnp.dot(x_ref[...], w_ref[0],
                             preferred_element_type=jnp.float32)
    return pl.pallas_call(
        body,
        grid=(GN, R + 1),
        in_specs=[
            pl.BlockSpec((BN, D), lambda i, r: (i, 0)),
            pl.BlockSpec((1, D, D), lambda i, r: (r, 0, 0)),
        ],
        out_specs=pl.BlockSpec((BN, D), lambda i, r: (r * GN + i, 0)),
        out_shape=jax.ShapeDtypeStruct(((R + 1) * N, D), jnp.float32),
    )(x, wall)


def _mm2(hfull1, p0, p1, b1, wall):
    def body(base_ref, p0_ref, p1_ref, b_ref, w_ref, o_ref):
        h = base_ref[...] + p0_ref[...] + p1_ref[...] + b_ref[...]
        h = jnp.maximum(h, 0.0)
        o_ref[...] = jnp.dot(h, w_ref[0], preferred_element_type=jnp.float32)
    return pl.pallas_call(
        body,
        grid=(GN, R + 1),
        in_specs=[
            pl.BlockSpec((BN, D), lambda i, r: (R * GN + i, 0)),
            pl.BlockSpec((BN, D), lambda i, r: (i, 0)),
            pl.BlockSpec((BN, D), lambda i, r: (i, 0)),
            pl.BlockSpec((1, D), lambda i, r: (0, 0)),
            pl.BlockSpec((1, D, D), lambda i, r: (r, 0, 0)),
        ],
        out_specs=pl.BlockSpec((BN, D), lambda i, r: (r * GN + i, 0)),
        out_shape=jax.ShapeDtypeStruct(((R + 1) * N, D), jnp.float32),
    )(hfull1, p0, p1, b1, wall)


def _combine(hfull2, p0, p1, b2):
    def body(base_ref, p0_ref, p1_ref, b_ref, o_ref):
        o_ref[...] = base_ref[...] + p0_ref[...] + p1_ref[...] + b_ref[...]
    return pl.pallas_call(
        body,
        grid=(GN,),
        in_specs=[
            pl.BlockSpec((BN, D), lambda i: (R * GN + i, 0)),
            pl.BlockSpec((BN, D), lambda i: (i, 0)),
            pl.BlockSpec((BN, D), lambda i: (i, 0)),
            pl.BlockSpec((1, D), lambda i: (0, 0)),
        ],
        out_specs=pl.BlockSpec((BN, D), lambda i: (i, 0)),
        out_shape=jax.ShapeDtypeStruct((N, D), jnp.float32),
    )(hfull2, p0, p1, b2)


def kernel(x, edge_index, edge_type, W1, root1, b1, W2, root2, b2):
    src = edge_index[0]
    dst = edge_index[1]
    zrows = jnp.zeros((RZC, D), jnp.float32)
    lanes = jnp.arange(D)
    oh = (lanes[None, :] == (jnp.arange(R) * L)[:, None]).astype(jnp.float32)
    sel = (lanes[None, :, None] ==
           (jnp.arange(R) * L)[:, None, None]).astype(jnp.float32)

    meta = _meta(edge_type.reshape(CHT, K), src.reshape(CHT, K),
                 dst.reshape(CHT, K)).reshape(CHT * MROW)
    hist = _hist_kernel(meta, oh, zrows)
    invrep = _invrep(hist[:N], hist[NPAD:NPAD + N], sel)
    rec = _prep_kernel(invrep, meta)

    wall1 = jnp.concatenate([W1, root1[None]], axis=0)
    wall2 = jnp.concatenate([W2, root2[None]], axis=0)

    hfull1 = _mm1(x, wall1)
    part1 = _edge_kernel(hfull1, meta, rec, zrows)
    hfull2 = _mm2(hfull1, part1[:N], part1[NPAD:NPAD + N],
                  b1.reshape(1, D), wall2)
    part2 = _edge_kernel(hfull2, meta, rec, zrows)
    return _combine(hfull2, part2[:N], part2[NPAD:NPAD + N], b2.reshape(1, D))


# speedup vs baseline: 22.0410x; 1.0632x over previous
import functools

import jax
import jax.numpy as jnp
from jax import lax
from jax.experimental import pallas as pl
from jax.experimental.pallas import tpu as pltpu
from jax.experimental.pallas import tpu_sc as plsc

N = 10000
E = 320000
R = 8
D = 128

NC = 2
NS = 16
NW = NC * NS
L = 16

EPW = E // NW
K = 80
NCH = EPW // K
KV = K // L

NPAD = 10240
RPT = NPAD // NS
RZC = 640

BN = 1000
GN = N // BN


def _mesh():
    return plsc.VectorSubcoreMesh(
        core_axis_name="c", subcore_axis_name="s",
        num_cores=NC, num_subcores=NS)


@functools.partial(
    pl.kernel,
    out_type=jax.ShapeDtypeStruct((NC * NPAD, D), jnp.float32),
    mesh=_mesh(),
    scratch_types=[
        pltpu.VMEM((240,), jnp.int32),
        pltpu.VMEM((240,), jnp.int32),
        pltpu.VMEM((240,), jnp.int32),
        pltpu.VMEM((K,), jnp.int32),
        pltpu.VMEM((K,), jnp.int32),
        pltpu.VMEM((K,), jnp.int32),
        pltpu.VMEM((K, D), jnp.float32),
        pltpu.VMEM((K, D), jnp.float32),
        pltpu.VMEM((K, D), jnp.float32),
        pltpu.VMEM_SHARED((R, D), jnp.float32),
        pltpu.VMEM_SHARED((NPAD, D), jnp.float32),
        pltpu.SemaphoreType.DMA,
        pltpu.SemaphoreType.DMA,
        pltpu.SemaphoreType.DMA,
        pltpu.SemaphoreType.DMA,
        pltpu.SemaphoreType.DMA,
        pltpu.SemaphoreType.DMA,
    ],
)
def _hist_kernel(meta_hbm, oh_hbm, zrows_hbm, out_hbm,
                 meta0, meta1, meta2, dst0, dst1, dst2, oh0, oh1, oh2,
                 oh_sh, hist_sh,
                 semg0, semg1, semg2, semc0, semc1, semc2):
    c = lax.axis_index("c")
    s = lax.axis_index("s")
    wid = c * NS + s
    meta_v = (meta0, meta1, meta2)
    dst_v = (dst0, dst1, dst2)
    oh_v = (oh0, oh1, oh2)
    semg = (semg0, semg1, semg2)
    semc = (semc0, semc1, semc2)

    @pl.when(s == 0)
    def _():
        pltpu.sync_copy(oh_hbm, oh_sh)
    for z in range(RPT // RZC):
        pltpu.sync_copy(zrows_hbm, hist_sh.at[pl.ds(s * RPT + z * RZC, RZC)])
    plsc.subcore_barrier()

    def wait_scat(b):
        pltpu.make_async_copy(oh_v[b], hist_sh.at[dst_v[b]],
                              semc[b]).wait()

    def issue(i, b, scat_pending):
        if scat_pending is not False:
            @pl.when(scat_pending)
            def _():
                wait_scat(b)
        row = wid * NCH + i
        pltpu.sync_copy(meta_hbm.at[pl.ds(row * MROW + K, 240)], meta_v[b])
        for q in range(KV):
            dst_v[b][pl.ds(q * L, L)] = meta_v[b][pl.ds(q * L, L)]
        pltpu.async_copy(oh_sh.at[meta_v[b].at[pl.ds(2 * K, K)]],
                         oh_v[b], semg[b])

    def finish(b):
        pltpu.make_async_copy(oh_sh.at[meta_v[b].at[pl.ds(2 * K, K)]],
                              oh_v[b], semg[b]).wait()
        pltpu.async_copy(oh_v[b], hist_sh.at[dst_v[b]], semc[b], add=True)

    issue(0, 0, False)
    issue(1, 1, False)

    def triple(t, carry):
        finish(0)
        issue(3 * t + 2, 2, t >= 1)
        finish(1)
        issue(3 * t + 3, 0, True)
        finish(2)
        issue(3 * t + 4, 1, True)
        return carry
    lax.fori_loop(0, (NCH - 2) // 3, triple, 0)

    finish(0)
    finish(1)
    wait_scat(2)
    wait_scat(0)
    wait_scat(1)
    plsc.subcore_barrier()
    for z in range(RPT // RZC):
        off = s * RPT + z * RZC
        pltpu.sync_copy(hist_sh.at[pl.ds(off, RZC)],
                        out_hbm.at[pl.ds(c * NPAD + off, RZC)])


def _invrep(h0, h1, sel):
    def body(h0_ref, h1_ref, sel_ref, o_ref):
        tot = h0_ref[...] + h1_ref[...]
        m = jnp.dot(tot, sel_ref[0], preferred_element_type=jnp.float32)
        o_ref[...] = jnp.broadcast_to(1.0 / jnp.maximum(m, 1.0), (BN, D))
    return pl.pallas_call(
        body,
        grid=(GN, R),
        in_specs=[
            pl.BlockSpec((BN, D), lambda i, r: (i, 0)),
            pl.BlockSpec((BN, D), lambda i, r: (i, 0)),
            pl.BlockSpec((1, D, 1), lambda i, r: (r, 0, 0)),
        ],
        out_specs=pl.BlockSpec((BN, D), lambda i, r: (r * GN + i, 0)),
        out_shape=jax.ShapeDtypeStruct((R * N, D), jnp.float32),
    )(h0, h1, sel)


MROW = 4 * K
CHT = E // K
BC = CHT // GN


def _meta(et2, src2, dst2):
    def body(t_ref, s_ref, d_ref, o_ref):
        t = t_ref[...]
        o_ref[:, 0:K] = t * N + s_ref[...]
        o_ref[:, K:2 * K] = d_ref[...]
        o_ref[:, 2 * K:3 * K] = t * N + d_ref[...]
        o_ref[:, 3 * K:4 * K] = t
    return pl.pallas_call(
        body,
        grid=(GN,),
        in_specs=[
            pl.BlockSpec((BC, K), lambda i: (i, 0)),
            pl.BlockSpec((BC, K), lambda i: (i, 0)),
            pl.BlockSpec((BC, K), lambda i: (i, 0)),
        ],
        out_specs=pl.BlockSpec((BC, MROW), lambda i: (i, 0)),
        out_shape=jax.ShapeDtypeStruct((CHT, MROW), jnp.int32),
    )(et2, src2, dst2)


MROW2 = K * L


@functools.partial(
    pl.kernel,
    out_type=jax.ShapeDtypeStruct((CHT * MROW2,), jnp.float32),
    mesh=_mesh(),
    scratch_types=[
        pltpu.VMEM((MROW,), jnp.int32),
        pltpu.VMEM((MROW,), jnp.int32),
        pltpu.VMEM((K, D), jnp.float32),
        pltpu.VMEM((K, D), jnp.float32),
        pltpu.VMEM((MROW2,), jnp.float32),
        pltpu.VMEM((MROW2,), jnp.float32),
        pltpu.SemaphoreType.DMA,
        pltpu.SemaphoreType.DMA,
    ],
)
def _prep_kernel(invrep_hbm, meta_hbm, out_hbm,
                 meta0, meta1, sc0, sc1, rec0, rec1, sem0, sem1):
    c = lax.axis_index("c")
    s = lax.axis_index("s")
    wid = c * NS + s
    meta_v = (meta0, meta1)
    sc_v = (sc0, sc1)
    rec_v = (rec0, rec1)
    sems = (sem0, sem1)

    def issue(i, b):
        row = wid * NCH + i
        pltpu.sync_copy(meta_hbm.at[pl.ds(row * MROW, MROW)], meta_v[b])
        pltpu.async_copy(
            invrep_hbm.at[meta_v[b].at[pl.ds(2 * K, K)]], sc_v[b], sems[b])

    def finish(i, b):
        pltpu.make_async_copy(
            invrep_hbm.at[meta_v[b].at[pl.ds(2 * K, K)]],
            sc_v[b], sems[b]).wait()

        def srow(j, carry2):
            rec_v[b][pl.ds(j * L, L)] = sc_v[b][j, pl.ds(0, L)]
            return carry2
        lax.fori_loop(0, K, srow, 0, unroll=4)
        row = wid * NCH + i
        pltpu.sync_copy(rec_v[b], out_hbm.at[pl.ds(row * MROW2, MROW2)])

    issue(0, 0)

    def pair(i, carry):
        c1 = 2 * i + 1

        @pl.when(c1 < NCH)
        def _():
            issue(c1, 1)
        finish(2 * i, 0)

        @pl.when(c1 + 1 < NCH)
        def _():
            issue(c1 + 1, 0)

        @pl.when(c1 < NCH)
        def _():
            finish(c1, 1)
        return carry
    lax.fori_loop(0, (NCH + 1) // 2, pair, 0)


NBUF = 4


def _buf_scratch():
    kinds = []
    kinds += [pltpu.VMEM((2 * K,), jnp.int32)] * NBUF
    kinds += [pltpu.VMEM((MROW2,), jnp.float32)] * NBUF
    kinds += [pltpu.VMEM((K,), jnp.int32)] * NBUF
    kinds += [pltpu.VMEM((K,), jnp.int32)] * NBUF
    kinds += [pltpu.VMEM((K, D), jnp.float32)] * NBUF
    kinds += [pltpu.VMEM_SHARED((NPAD, D), jnp.float32)]
    kinds += [pltpu.SemaphoreType.DMA] * (4 * NBUF)
    return kinds


@functools.partial(
    pl.kernel,
    out_type=jax.ShapeDtypeStruct((NC * NPAD, D), jnp.float32),
    mesh=_mesh(),
    scratch_types=_buf_scratch(),
)
def _edge_kernel(h_hbm, meta_hbm, rec_hbm, zrows_hbm, out_hbm, *scr):
    c = lax.axis_index("c")
    s = lax.axis_index("s")
    wid = c * NS + s
    meta_v = scr[0:NBUF]
    rec_v = scr[NBUF:2 * NBUF]
    gidx_v = scr[2 * NBUF:3 * NBUF]
    dst_v = scr[3 * NBUF:4 * NBUF]
    rows_v = scr[4 * NBUF:5 * NBUF]
    acc_sh = scr[5 * NBUF]
    semm = scr[5 * NBUF + 1:5 * NBUF + 1 + NBUF]
    semr = scr[5 * NBUF + 1 + NBUF:5 * NBUF + 1 + 2 * NBUF]
    semg = scr[5 * NBUF + 1 + 2 * NBUF:5 * NBUF + 1 + 3 * NBUF]
    semc = scr[5 * NBUF + 1 + 3 * NBUF:5 * NBUF + 1 + 4 * NBUF]

    pltpu.sync_copy(zrows_hbm, acc_sh.at[pl.ds(s * RPT, RPT)])
    plsc.subcore_barrier()

    def wait_scat(b):
        pltpu.make_async_copy(rows_v[b], acc_sh.at[dst_v[b]],
                              semc[b]).wait()

    def prefetch(i, b):
        row = wid * NCH + i
        pltpu.async_copy(meta_hbm.at[pl.ds(row * MROW, 2 * K)],
                         meta_v[b], semm[b])
        pltpu.async_copy(rec_hbm.at[pl.ds(row * MROW2, MROW2)],
                         rec_v[b], semr[b])

    def launch(i, b, scat_pending):
        if scat_pending is not False:
            @pl.when(scat_pending)
            def _():
                wait_scat(b)
        row = wid * NCH + i
        pltpu.make_async_copy(meta_hbm.at[pl.ds(row * MROW, 2 * K)],
                              meta_v[b], semm[b]).wait()
        for q in range(KV):
            sl = pl.ds(q * L, L)
            gidx_v[b][sl] = meta_v[b][pl.ds(q * L, L)]
            dst_v[b][sl] = meta_v[b][pl.ds(K + q * L, L)]
        pltpu.async_copy(h_hbm.at[gidx_v[b]], rows_v[b], semg[b])

    def finish(b):
        pltpu.make_async_copy(rec_hbm.at[pl.ds(0, MROW2)],
                              rec_v[b], semr[b]).wait()
        pltpu.make_async_copy(h_hbm.at[gidx_v[b]], rows_v[b],
                              semg[b]).wait()

        def srow(j, carry2):
            sv = rec_v[b][pl.ds(j * L, L)]
            for cp in range(D // L):
                sl = pl.ds(cp * L, L)
                rows_v[b][j, sl] = rows_v[b][j, sl] * sv
            return carry2
        lax.fori_loop(0, K, srow, 0, unroll=4)
        pltpu.async_copy(rows_v[b], acc_sh.at[dst_v[b]], semc[b], add=True)

    prefetch(0, 0)
    prefetch(1, 1)
    launch(0, 0, False)
    prefetch(2, 2)
    launch(1, 1, False)

    def quad(t, carry):
        for off in range(4):
            b = off
            finish(b)
            launch(4 * t + off + 2, (off + 2) % NBUF,
                   t >= 1 if off < 2 else True)
            prefetch(4 * t + off + 3, (off + 3) % NBUF)
        return carry
    lax.fori_loop(0, (NCH - 5) // 4, quad, 0)

    base = 4 * ((NCH - 5) // 4)
    for k in range(base, NCH):
        b = k % NBUF
        finish(b)
        if k + 2 < NCH:
            launch(k + 2, (k + 2) % NBUF, True)
        if k + 3 < NCH:
            prefetch(k + 3, (k + 3) % NBUF)
    for k in range(NCH - NBUF, NCH):
        wait_scat(k % NBUF)
    plsc.subcore_barrier()
    pltpu.sync_copy(acc_sh.at[pl.ds(s * RPT, RPT)],
                    out_hbm.at[pl.ds(c * NPAD + s * RPT, RPT)])


def _mm1(x, wall):
    def body(x_ref, w_ref, o_ref):
        o_ref[...] = jnp.dot(x_ref[...], w_ref[0],
                             preferred_element_type=jnp.float32)
    return pl.pallas_call(
        body,
        grid=(GN, R + 1),
        in_specs=[
            pl.BlockSpec((BN, D), lambda i, r: (i, 0)),
            pl.BlockSpec((1, D, D), lambda i, r: (r, 0, 0)),
        ],
        out_specs=pl.BlockSpec((BN, D), lambda i, r: (r * GN + i, 0)),
        out_shape=jax.ShapeDtypeStruct(((R + 1) * N, D), jnp.float32),
    )(x, wall)


def _mm2(hfull1, p0, p1, b1, wall):
    def body(base_ref, p0_ref, p1_ref, b_ref, w_ref, o_ref):
        h = base_ref[...] + p0_ref[...] + p1_ref[...] + b_ref[...]
        h = jnp.maximum(h, 0.0)
        o_ref[...] = jnp.dot(h, w_ref[0], preferred_element_type=jnp.float32)
    return pl.pallas_call(
        body,
        grid=(GN, R + 1),
        in_specs=[
            pl.BlockSpec((BN, D), lambda i, r: (R * GN + i, 0)),
            pl.BlockSpec((BN, D), lambda i, r: (i, 0)),
            pl.BlockSpec((BN, D), lambda i, r: (i, 0)),
            pl.BlockSpec((1, D), lambda i, r: (0, 0)),
            pl.BlockSpec((1, D, D), lambda i, r: (r, 0, 0)),
        ],
        out_specs=pl.BlockSpec((BN, D), lambda i, r: (r * GN + i, 0)),
        out_shape=jax.ShapeDtypeStruct(((R + 1) * N, D), jnp.float32),
    )(hfull1, p0, p1, b1, wall)


def _combine(hfull2, p0, p1, b2):
    def body(base_ref, p0_ref, p1_ref, b_ref, o_ref):
        o_ref[...] = base_ref[...] + p0_ref[...] + p1_ref[...] + b_ref[...]
    return pl.pallas_call(
        body,
        grid=(GN,),
        in_specs=[
            pl.BlockSpec((BN, D), lambda i: (R * GN + i, 0)),
            pl.BlockSpec((BN, D), lambda i: (i, 0)),
            pl.BlockSpec((BN, D), lambda i: (i, 0)),
            pl.BlockSpec((1, D), lambda i: (0, 0)),
        ],
        out_specs=pl.BlockSpec((BN, D), lambda i: (i, 0)),
        out_shape=jax.ShapeDtypeStruct((N, D), jnp.float32),
    )(hfull2, p0, p1, b2)


def kernel(x, edge_index, edge_type, W1, root1, b1, W2, root2, b2):
    src = edge_index[0]
    dst = edge_index[1]
    zrows = jnp.zeros((RZC, D), jnp.float32)
    lanes = jnp.arange(D)
    oh = (lanes[None, :] == (jnp.arange(R) * L)[:, None]).astype(jnp.float32)
    sel = (lanes[None, :, None] ==
           (jnp.arange(R) * L)[:, None, None]).astype(jnp.float32)

    meta = _meta(edge_type.reshape(CHT, K), src.reshape(CHT, K),
                 dst.reshape(CHT, K)).reshape(CHT * MROW)
    hist = _hist_kernel(meta, oh, zrows)
    invrep = _invrep(hist[:N], hist[NPAD:NPAD + N], sel)
    rec = _prep_kernel(invrep, meta)

    wall1 = jnp.concatenate([W1, root1[None]], axis=0)
    wall2 = jnp.concatenate([W2, root2[None]], axis=0)

    hfull1 = _mm1(x, wall1)
    part1 = _edge_kernel(hfull1, meta, rec, zrows)
    hfull2 = _mm2(hfull1, part1[:N], part1[NPAD:NPAD + N],
                  b1.reshape(1, D), wall2)
    part2 = _edge_kernel(hfull2, meta, rec, zrows)
    return _combine(hfull2, part2[:N], part2[NPAD:NPAD + N], b2.reshape(1, D))
